# Initial kernel scaffold; baseline (speedup 1.0000x reference)
import jax
import jax.numpy as jnp
from jax.experimental import pallas as pl


def kernel(x, edge_index, edge_attr, mask, params):
    raise NotImplementedError("write your pallas kernel here")


# speedup vs baseline: 5.6232x; 5.6232x over previous
import functools

import jax
import jax.numpy as jnp
from jax import lax
from jax.experimental import pallas as pl
from jax.experimental.pallas import tpu as pltpu
from jax.experimental.pallas import tpu_sc as plsc

N = 10000
E = 320000
HID = 128
T_IN = 8
T_OUT = 4
ALPHA = 0.5
D_DEPTH = 2
NG = N // 10

NC = 2
NS = 16
NW = NC * NS
CH = 128
K_CH = (E + NW * CH - 1) // (NW * CH)
E_PAD = NW * CH * K_CH
N_PAD = 10240
SLAB = 632
SLAB_LAST = N - 15 * SLAB


def _agg_body(u_hbm, srcr_hbm, dstr_hbm, out_hbm, src_v, dst_v, rows_v, acc_sh, sem):
    c = lax.axis_index("c")
    s = lax.axis_index("s")
    wid = c * NS + s
    @pl.when(s < NS - 1)
    def _():
        pltpu.sync_copy(u_hbm.at[pl.ds(s * SLAB, SLAB)],
                        acc_sh.at[pl.ds(s * SLAB, SLAB)])

    @pl.when(s == NS - 1)
    def _():
        pltpu.sync_copy(u_hbm.at[pl.ds(15 * SLAB, SLAB_LAST)],
                        acc_sh.at[pl.ds(15 * SLAB, SLAB_LAST)])

    pltpu.sync_copy(srcr_hbm.at[wid], src_v)
    pltpu.sync_copy(dstr_hbm.at[wid], dst_v)
    plsc.subcore_barrier()

    def body(j, carry):
        pltpu.async_copy(u_hbm.at[src_v.at[j]], rows_v, sem).wait()
        pltpu.sync_copy(rows_v, acc_sh.at[dst_v.at[j]], add=True)
        return carry

    lax.fori_loop(0, K_CH, body, 0)
    plsc.subcore_barrier()

    @pl.when(s < NS - 1)
    def _():
        pltpu.sync_copy(acc_sh.at[pl.ds(s * SLAB, SLAB)],
                        out_hbm.at[c, pl.ds(s * SLAB, SLAB)])

    @pl.when(s == NS - 1)
    def _():
        pltpu.sync_copy(acc_sh.at[pl.ds(15 * SLAB, SLAB_LAST)],
                        out_hbm.at[c, pl.ds(15 * SLAB, SLAB_LAST)])


_agg = pl.kernel(
    _agg_body,
    out_type=jax.ShapeDtypeStruct((NC, N, HID), jnp.float32),
    mesh=plsc.VectorSubcoreMesh(core_axis_name="c", subcore_axis_name="s",
                                num_cores=NC, num_subcores=NS),
    scratch_types=[
        pltpu.VMEM((K_CH, CH), jnp.int32),
        pltpu.VMEM((K_CH, CH), jnp.int32),
        pltpu.VMEM((CH, HID), jnp.float32),
        pltpu.VMEM_SHARED((N_PAD, HID), jnp.float32),
        pltpu.SemaphoreType.DMA,
    ],
)


RB = 2000


def _bcast_spec(shape):
    return pl.BlockSpec(shape, lambda i: (0,) * len(shape))


def _row_spec(cols):
    return pl.BlockSpec((RB, cols), lambda i: (i, 0))


def _ghost_kernel(xin_ref, w1t, b1, w2t, g1t, gb1, g2t, out_ref):
    xin = xin_ref[...]
    z = jnp.maximum(jnp.dot(xin, w1t[...]) + b1[...], 0.0)
    delta = jnp.dot(z, w2t[...])
    g = jnp.maximum(jnp.dot(xin, g1t[...]) + gb1[...], 0.0)
    gate = jax.nn.sigmoid(jnp.dot(g, g2t[...]))
    out_ref[...] = delta * gate


def _embed_kernel(x_ref, dinv_ref, linw, linb, w0t, out_ref):
    h0 = jnp.maximum(x_ref[...] * linw[...] + linb[...], 0.0)
    out_ref[...] = jnp.dot(dinv_ref[...] * h0, w0t[...])


def _mid_kernel(sa_ref, sb_ref, u_ref, dinv_ref, b0, w1t, out_ref):
    pre = dinv_ref[...] * (sa_ref[...] + sb_ref[...] - u_ref[...]) + b0[...]
    h = jnp.maximum(pre, 0.0)
    out_ref[...] = jnp.dot(dinv_ref[...] * h, w1t[...])


def _gru_kernel(sa_ref, sb_ref, u_ref, dinv_ref, b1, wiht, whht, bih, bhh,
                h_ref, out_ref):
    g = jnp.maximum(dinv_ref[...] * (sa_ref[...] + sb_ref[...] - u_ref[...])
                    + b1[...], 0.0)
    gi = jnp.dot(g, wiht[...]) + bih[...]
    gh = jnp.dot(h_ref[...], whht[...]) + bhh[...]
    ir, iz, inn = gi[:, :HID], gi[:, HID:2 * HID], gi[:, 2 * HID:]
    hr, hz, hn = gh[:, :HID], gh[:, HID:2 * HID], gh[:, 2 * HID:]
    r = jax.nn.sigmoid(ir + hr)
    z = jax.nn.sigmoid(iz + hz)
    n = jnp.tanh(inn + r * hn)
    out_ref[...] = (1.0 - z) * n + z * h_ref[...]


def _ln(x, g, b):
    m = jnp.mean(x, axis=-1, keepdims=True)
    v = jnp.mean((x - m) ** 2, axis=-1, keepdims=True)
    return (x - m) / jnp.sqrt(v + 1e-05) * g + b


def _pred_kernel(h_ref, w1t, pb1, g1, bb1, w2t, pb2, g2, bb2, p3, pb3,
                 linw, linb, w0t, dinv_ref, y_ref, u0_ref):
    z1 = jnp.maximum(_ln(jnp.dot(h_ref[...], w1t[...]) + pb1[...],
                         g1[...], bb1[...]), 0.0)
    z2 = jnp.maximum(_ln(jnp.dot(z1, w2t[...]) + pb2[...],
                         g2[...], bb2[...]), 0.0)
    y = jnp.sum(z2 * p3[...], axis=1, keepdims=True) + pb3[...]
    y_ref[...] = y
    h0 = jnp.maximum(y * linw[...] + linb[...], 0.0)
    u0_ref[...] = jnp.dot(dinv_ref[...] * h0, w0t[...])


_ghost_call = pl.pallas_call(
    _ghost_kernel,
    out_shape=jax.ShapeDtypeStruct((T_IN * NG, 128), jnp.float32),
    in_specs=[_bcast_spec((T_IN * NG, 128)), _bcast_spec((128, 128)),
              _bcast_spec((1, 128)), _bcast_spec((128, 128)),
              _bcast_spec((128, 128)), _bcast_spec((1, 128)),
              _bcast_spec((128, 128))],
    out_specs=_bcast_spec((T_IN * NG, 128)),
    grid=(1,),
)

_embed_call = pl.pallas_call(
    _embed_kernel,
    out_shape=jax.ShapeDtypeStruct((T_IN * N, HID), jnp.float32),
    in_specs=[_row_spec(1), _row_spec(1), _bcast_spec((1, HID)),
              _bcast_spec((1, HID)), _bcast_spec((HID, HID))],
    out_specs=_row_spec(HID),
    grid=(T_IN * N // RB,),
)

_mid_call = pl.pallas_call(
    _mid_kernel,
    out_shape=jax.ShapeDtypeStruct((N, HID), jnp.float32),
    in_specs=[_row_spec(HID), _row_spec(HID), _row_spec(HID), _row_spec(1),
              _bcast_spec((1, HID)), _bcast_spec((HID, HID))],
    out_specs=_row_spec(HID),
    grid=(N // RB,),
)

_gru_call = pl.pallas_call(
    _gru_kernel,
    out_shape=jax.ShapeDtypeStruct((N, HID), jnp.float32),
    in_specs=[_row_spec(HID), _row_spec(HID), _row_spec(HID), _row_spec(1),
              _bcast_spec((1, HID)), _bcast_spec((HID, 3 * HID)),
              _bcast_spec((HID, 3 * HID)), _bcast_spec((1, 3 * HID)),
              _bcast_spec((1, 3 * HID)), _row_spec(HID)],
    out_specs=_row_spec(HID),
    grid=(N // RB,),
)

_pred_call = pl.pallas_call(
    _pred_kernel,
    out_shape=(jax.ShapeDtypeStruct((N, 1), jnp.float32),
               jax.ShapeDtypeStruct((N, HID), jnp.float32)),
    in_specs=[_row_spec(HID),
              _bcast_spec((HID, HID)), _bcast_spec((1, HID)),
              _bcast_spec((1, HID)), _bcast_spec((1, HID)),
              _bcast_spec((HID, HID)), _bcast_spec((1, HID)),
              _bcast_spec((1, HID)), _bcast_spec((1, HID)),
              _bcast_spec((1, HID)), _bcast_spec((1, 1)),
              _bcast_spec((1, HID)), _bcast_spec((1, HID)),
              _bcast_spec((HID, HID)), _row_spec(1)],
    out_specs=(_row_spec(1), _row_spec(HID)),
    grid=(N // RB,),
)


def _pad_w(w, shape):
    out = jnp.zeros(shape, jnp.float32)
    return out.at[:w.shape[0], :w.shape[1]].set(w)


def kernel(x, edge_index, edge_attr, mask, params):
    p = params
    src, dst = edge_index[0], edge_index[1]

    deg = 1.0 + jnp.zeros((N,), jnp.float32).at[dst].add(1.0)
    dinv = 1.0 / jnp.sqrt(deg)
    dinv2 = dinv[:, None]

    pos = jnp.arange(E, dtype=src.dtype)
    pos_min = jnp.full((N,), E, dtype=src.dtype).at[src].min(pos)
    has_first = pos_min < E
    first_pos = jnp.minimum(pos_min, E - 1)
    first_out = jnp.where(has_first, dst[first_pos], jnp.array(-1, src.dtype))
    ghost_idx = jnp.arange(0, N, 10)
    dK = []
    cur = first_out[ghost_idx]
    for _ in range(D_DEPTH):
        nxt = first_out[cur]
        nxt = jnp.where(nxt < 0, cur, nxt)
        dK.append(nxt)
        cur = nxt
    dist = jnp.maximum(edge_attr[:, 0], 1e-06)
    first_dx = jnp.where(has_first, dist[first_pos], jnp.float32(-1.0))
    dx_bnd = jnp.maximum(first_dx[ghost_idx], 1e-06)

    pad = E_PAD - E
    src_r = jnp.concatenate([src, jnp.zeros((pad,), src.dtype)]).reshape(NW, K_CH, CH)
    dst_r = jnp.concatenate([dst, jnp.full((pad,), N, dst.dtype)]).reshape(NW, K_CH, CH)

    X = x[:, :, 0]
    xg = X[0::10]
    d1 = X[dK[0]]
    d2 = X[dK[1]]
    xin = jnp.stack([xg.T, d1.T, d2.T,
                     jnp.broadcast_to(dx_bnd[None, :], (T_IN, NG))], axis=-1)
    xin = xin.reshape(T_IN * NG, 4)
    xin_p = jnp.concatenate([xin, jnp.zeros((T_IN * NG, 124), jnp.float32)], axis=1)
    gf_out = _ghost_call(
        xin_p,
        _pad_w(p['gf_fc1_W'], (128, 128)).T,
        _pad_w(p['gf_fc1_b'][None, :], (1, 128)),
        _pad_w(p['gf_fc2_W'], (128, 128)).T,
        _pad_w(p['gf_g1_W'], (128, 128)).T,
        _pad_w(p['gf_g1_b'][None, :], (1, 128)),
        _pad_w(p['gf_g2_W'], (128, 128)).T,
    )
    dcol = gf_out[:, 0].reshape(T_IN, NG)
    fused = xg.T + ALPHA * dcol
    Xr = X.reshape(NG, 10, T_IN)
    Xf = jnp.concatenate([fused.T[:, None, :], Xr[:, 1:, :]], axis=1).reshape(N, T_IN)

    linw = p['lin_W'][:, 0][None, :]
    linb = p['lin_b'][None, :]
    w0t = p['conv0_W'].T
    w1t = p['conv1_W'].T
    U0_all = _embed_call(Xf.T.reshape(T_IN * N, 1),
                         jnp.tile(dinv, T_IN)[:, None], linw, linb, w0t)

    wiht = p['gru_Wih'].T
    whht = p['gru_Whh'].T
    bih = p['gru_bih'][None, :]
    bhh = p['gru_bhh'][None, :]
    b0 = p['conv0_b'][None, :]
    b1 = p['conv1_b'][None, :]

    def step(u0, h):
        s0 = _agg(u0, src_r, dst_r)
        u1 = _mid_call(s0[0], s0[1], u0, dinv2, b0, w1t)
        s1 = _agg(u1, src_r, dst_r)
        return _gru_call(s1[0], s1[1], u1, dinv2, b1, wiht, whht, bih, bhh, h)

    h = jnp.zeros((N, HID), jnp.float32)
    for t in range(T_IN):
        h = step(lax.dynamic_slice_in_dim(U0_all, t * N, N, 0), h)

    pred_args = (p['pred_W1'].T, p['pred_b1'][None, :], p['pred_ln1_g'][None, :],
                 p['pred_ln1_b'][None, :], p['pred_W2'].T, p['pred_b2'][None, :],
                 p['pred_ln2_g'][None, :], p['pred_ln2_b'][None, :],
                 p['pred_W3'], p['pred_b3'][None, :], linw, linb, w0t)

    outs = []
    u0 = U0_all[(T_IN - 1) * N:]
    for k in range(T_OUT):
        h = step(u0, h)
        y, u0 = _pred_call(h, *pred_args, dinv2)
        outs.append(y.reshape(NG, 10)[:, 1:].reshape(-1, 1))
    return jnp.concatenate(outs, axis=1)


# speedup vs baseline: 5.9621x; 1.0603x over previous
import functools

import jax
import jax.numpy as jnp
from jax import lax
from jax.experimental import pallas as pl
from jax.experimental.pallas import tpu as pltpu
from jax.experimental.pallas import tpu_sc as plsc

N = 10000
E = 320000
HID = 128
HH = HID // 2
T_IN = 8
T_OUT = 4
ALPHA = 0.5
D_DEPTH = 2
NG = N // 10

NC = 2
NS = 16
CH = 128
NBUF = 4
K_CH = -(-E // (NS * CH * NBUF)) * NBUF
E_PAD = NS * CH * K_CH
N_PAD = 10240
SLAB = 632
SLAB_LAST = N - 15 * SLAB


def _agg_body(u2_hbm, srcr_hbm, dstr_hbm, out_hbm, src_v, dst_v, rows_v,
              acc_sh, *sems):
    gsems = sems[:NBUF]
    ssems = sems[NBUF:]
    c = lax.axis_index("c")
    s = lax.axis_index("s")
    @pl.when(s < NS - 1)
    def _():
        pltpu.sync_copy(u2_hbm.at[pl.ds(c * N + s * SLAB, SLAB)],
                        acc_sh.at[pl.ds(s * SLAB, SLAB)])

    @pl.when(s == NS - 1)
    def _():
        pltpu.sync_copy(u2_hbm.at[pl.ds(c * N + 15 * SLAB, SLAB_LAST)],
                        acc_sh.at[pl.ds(15 * SLAB, SLAB_LAST)])

    pltpu.sync_copy(srcr_hbm.at[c, s], src_v)
    pltpu.sync_copy(dstr_hbm.at[s], dst_v)
    plsc.subcore_barrier()

    for b in range(NBUF):
        pltpu.async_copy(u2_hbm.at[src_v.at[b]], rows_v.at[b], gsems[b])

    def group(gi, carry):
        base = gi * NBUF
        for b in range(NBUF):
            j = base + b
            pltpu.make_async_copy(u2_hbm.at[src_v.at[j]], rows_v.at[b],
                                  gsems[b]).wait()
            pltpu.async_copy(rows_v.at[b], acc_sh.at[dst_v.at[j]], ssems[b],
                             add=True)
        for b in range(NBUF):
            jn = base + NBUF + b

            @pl.when(jn < K_CH)
            def _():
                pltpu.make_async_copy(rows_v.at[b],
                                      acc_sh.at[dst_v.at[base + b]],
                                      ssems[b]).wait()
                pltpu.async_copy(u2_hbm.at[src_v.at[jn]], rows_v.at[b],
                                 gsems[b])
        return carry

    lax.fori_loop(0, K_CH // NBUF, group, 0)
    for b in range(NBUF):
        pltpu.make_async_copy(rows_v.at[b],
                              acc_sh.at[dst_v.at[K_CH - NBUF + b]],
                              ssems[b]).wait()
    plsc.subcore_barrier()

    @pl.when(s < NS - 1)
    def _():
        pltpu.sync_copy(acc_sh.at[pl.ds(s * SLAB, SLAB)],
                        out_hbm.at[c, pl.ds(s * SLAB, SLAB)])

    @pl.when(s == NS - 1)
    def _():
        pltpu.sync_copy(acc_sh.at[pl.ds(15 * SLAB, SLAB_LAST)],
                        out_hbm.at[c, pl.ds(15 * SLAB, SLAB_LAST)])


_agg = pl.kernel(
    _agg_body,
    out_type=jax.ShapeDtypeStruct((NC, N, HH), jnp.float32),
    mesh=plsc.VectorSubcoreMesh(core_axis_name="c", subcore_axis_name="s",
                                num_cores=NC, num_subcores=NS),
    scratch_types=[
        pltpu.VMEM((K_CH, CH), jnp.int32),
        pltpu.VMEM((K_CH, CH), jnp.int32),
        pltpu.VMEM((NBUF, CH, HH), jnp.float32),
        pltpu.VMEM_SHARED((N_PAD, HH), jnp.float32),
    ] + [pltpu.SemaphoreType.DMA] * (2 * NBUF),
    compiler_params=pltpu.CompilerParams(use_tc_tiling_on_sc=False),
)


RB = 2000
NB = N // RB


def _bcast_spec(shape):
    return pl.BlockSpec(shape, lambda i: (0,) * len(shape))


def _row_spec(cols):
    return pl.BlockSpec((RB, cols), lambda i: (i, 0))


def _split_spec():
    return pl.BlockSpec((NC, RB, HH), lambda i: (0, i, 0))


def _split(u):
    return jnp.stack([u[:, :HH], u[:, HH:]], axis=0)


def _ghost_kernel(xin_ref, w1t, b1, w2t, g1t, gb1, g2t, out_ref):
    xin = xin_ref[...]
    z = jnp.maximum(jnp.dot(xin, w1t[...]) + b1[...], 0.0)
    delta = jnp.dot(z, w2t[...])
    g = jnp.maximum(jnp.dot(xin, g1t[...]) + gb1[...], 0.0)
    gate = jax.nn.sigmoid(jnp.dot(g, g2t[...]))
    out_ref[...] = delta * gate


def _embed_kernel(x_ref, dinv_ref, linw, linb, w0t, out_ref):
    h0 = jnp.maximum(x_ref[...] * linw[...] + linb[...], 0.0)
    out_ref[...] = _split(jnp.dot(dinv_ref[...] * h0, w0t[...]))[None]


def _mid_kernel(s0_ref, dinv_ref, b0, w1t, out_ref):
    sfull = jnp.concatenate([s0_ref[0], s0_ref[1]], axis=1)
    h = jnp.maximum(dinv_ref[...] * sfull + b0[...], 0.0)
    out_ref[...] = _split(jnp.dot(dinv_ref[...] * h, w1t[...]))


def _gru_kernel(s1_ref, dinv_ref, b1, wiht, whht, bih, bhh, h_ref, out_ref):
    sfull = jnp.concatenate([s1_ref[0], s1_ref[1]], axis=1)
    g = jnp.maximum(dinv_ref[...] * sfull + b1[...], 0.0)
    gi = jnp.dot(g, wiht[...]) + bih[...]
    gh = jnp.dot(h_ref[...], whht[...]) + bhh[...]
    ir, iz, inn = gi[:, :HID], gi[:, HID:2 * HID], gi[:, 2 * HID:]
    hr, hz, hn = gh[:, :HID], gh[:, HID:2 * HID], gh[:, 2 * HID:]
    r = jax.nn.sigmoid(ir + hr)
    z = jax.nn.sigmoid(iz + hz)
    n = jnp.tanh(inn + r * hn)
    out_ref[...] = (1.0 - z) * n + z * h_ref[...]


def _ln(x, g, b):
    m = jnp.mean(x, axis=-1, keepdims=True)
    v = jnp.mean((x - m) ** 2, axis=-1, keepdims=True)
    return (x - m) / jnp.sqrt(v + 1e-05) * g + b


def _pred_kernel(h_ref, w1t, pb1, g1, bb1, w2t, pb2, g2, bb2, p3, pb3,
                 linw, linb, w0t, dinv_ref, y_ref, u0_ref):
    z1 = jnp.maximum(_ln(jnp.dot(h_ref[...], w1t[...]) + pb1[...],
                         g1[...], bb1[...]), 0.0)
    z2 = jnp.maximum(_ln(jnp.dot(z1, w2t[...]) + pb2[...],
                         g2[...], bb2[...]), 0.0)
    y = jnp.sum(z2 * p3[...], axis=1, keepdims=True) + pb3[...]
    y_ref[...] = y
    h0 = jnp.maximum(y * linw[...] + linb[...], 0.0)
    u0_ref[...] = _split(jnp.dot(dinv_ref[...] * h0, w0t[...]))


_ghost_call = pl.pallas_call(
    _ghost_kernel,
    out_shape=jax.ShapeDtypeStruct((T_IN * NG, 128), jnp.float32),
    in_specs=[_bcast_spec((T_IN * NG, 128)), _bcast_spec((128, 128)),
              _bcast_spec((1, 128)), _bcast_spec((128, 128)),
              _bcast_spec((128, 128)), _bcast_spec((1, 128)),
              _bcast_spec((128, 128))],
    out_specs=_bcast_spec((T_IN * NG, 128)),
    grid=(1,),
)

_embed_call = pl.pallas_call(
    _embed_kernel,
    out_shape=jax.ShapeDtypeStruct((T_IN, NC, N, HH), jnp.float32),
    in_specs=[pl.BlockSpec((RB, 1), lambda t, i: (t * NB + i, 0)),
              pl.BlockSpec((RB, 1), lambda t, i: (i, 0)),
              pl.BlockSpec((1, HID), lambda t, i: (0, 0)),
              pl.BlockSpec((1, HID), lambda t, i: (0, 0)),
              pl.BlockSpec((HID, HID), lambda t, i: (0, 0))],
    out_specs=pl.BlockSpec((1, NC, RB, HH), lambda t, i: (t, 0, i, 0)),
    grid=(T_IN, NB),
)

_mid_call = pl.pallas_call(
    _mid_kernel,
    out_shape=jax.ShapeDtypeStruct((NC, N, HH), jnp.float32),
    in_specs=[_split_spec(), _row_spec(1),
              _bcast_spec((1, HID)), _bcast_spec((HID, HID))],
    out_specs=_split_spec(),
    grid=(NB,),
)

_gru_call = pl.pallas_call(
    _gru_kernel,
    out_shape=jax.ShapeDtypeStruct((N, HID), jnp.float32),
    in_specs=[_split_spec(), _row_spec(1),
              _bcast_spec((1, HID)), _bcast_spec((HID, 3 * HID)),
              _bcast_spec((HID, 3 * HID)), _bcast_spec((1, 3 * HID)),
              _bcast_spec((1, 3 * HID)), _row_spec(HID)],
    out_specs=_row_spec(HID),
    grid=(NB,),
)

_pred_call = pl.pallas_call(
    _pred_kernel,
    out_shape=(jax.ShapeDtypeStruct((N, 1), jnp.float32),
               jax.ShapeDtypeStruct((NC, N, HH), jnp.float32)),
    in_specs=[_row_spec(HID),
              _bcast_spec((HID, HID)), _bcast_spec((1, HID)),
              _bcast_spec((1, HID)), _bcast_spec((1, HID)),
              _bcast_spec((HID, HID)), _bcast_spec((1, HID)),
              _bcast_spec((1, HID)), _bcast_spec((1, HID)),
              _bcast_spec((1, HID)), _bcast_spec((1, 1)),
              _bcast_spec((1, HID)), _bcast_spec((1, HID)),
              _bcast_spec((HID, HID)), _row_spec(1)],
    out_specs=(_row_spec(1), _split_spec()),
    grid=(NB,),
)


def _pad_w(w, shape):
    out = jnp.zeros(shape, jnp.float32)
    return out.at[:w.shape[0], :w.shape[1]].set(w)


def kernel(x, edge_index, edge_attr, mask, params):
    p = params
    src, dst = edge_index[0], edge_index[1]

    deg = 1.0 + jnp.zeros((N,), jnp.float32).at[dst].add(1.0)
    dinv = 1.0 / jnp.sqrt(deg)
    dinv2 = dinv[:, None]

    pos = jnp.arange(E, dtype=src.dtype)
    pos_min = jnp.full((N,), E, dtype=src.dtype).at[src].min(pos)
    has_first = pos_min < E
    first_pos = jnp.minimum(pos_min, E - 1)
    first_out = jnp.where(has_first, dst[first_pos], jnp.array(-1, src.dtype))
    ghost_idx = jnp.arange(0, N, 10)
    dK = []
    cur = first_out[ghost_idx]
    for _ in range(D_DEPTH):
        nxt = first_out[cur]
        nxt = jnp.where(nxt < 0, cur, nxt)
        dK.append(nxt)
        cur = nxt
    dist = jnp.maximum(edge_attr[:, 0], 1e-06)
    first_dx = jnp.where(has_first, dist[first_pos], jnp.float32(-1.0))
    dx_bnd = jnp.maximum(first_dx[ghost_idx], 1e-06)

    pad = E_PAD - E
    src_r = jnp.concatenate([src, jnp.zeros((pad,), src.dtype)]).reshape(NS, K_CH, CH)
    src_all = jnp.stack([src_r, src_r + N], axis=0)
    dst_r = jnp.concatenate([dst, jnp.full((pad,), N, dst.dtype)]).reshape(NS, K_CH, CH)

    X = x[:, :, 0]
    xg = X[0::10]
    d1 = X[dK[0]]
    d2 = X[dK[1]]
    xin = jnp.stack([xg.T, d1.T, d2.T,
                     jnp.broadcast_to(dx_bnd[None, :], (T_IN, NG))], axis=-1)
    xin = xin.reshape(T_IN * NG, 4)
    xin_p = jnp.concatenate([xin, jnp.zeros((T_IN * NG, 124), jnp.float32)], axis=1)
    gf_out = _ghost_call(
        xin_p,
        _pad_w(p['gf_fc1_W'], (128, 128)).T,
        _pad_w(p['gf_fc1_b'][None, :], (1, 128)),
        _pad_w(p['gf_fc2_W'], (128, 128)).T,
        _pad_w(p['gf_g1_W'], (128, 128)).T,
        _pad_w(p['gf_g1_b'][None, :], (1, 128)),
        _pad_w(p['gf_g2_W'], (128, 128)).T,
    )
    dcol = gf_out[:, 0].reshape(T_IN, NG)
    fused = xg.T + ALPHA * dcol
    Xr = X.reshape(NG, 10, T_IN)
    Xf = jnp.concatenate([fused.T[:, None, :], Xr[:, 1:, :]], axis=1).reshape(N, T_IN)

    linw = p['lin_W'][:, 0][None, :]
    linb = p['lin_b'][None, :]
    w0t = p['conv0_W'].T
    w1t = p['conv1_W'].T
    U0_all = _embed_call(Xf.T.reshape(T_IN * N, 1), dinv2, linw, linb, w0t)

    wiht = p['gru_Wih'].T
    whht = p['gru_Whh'].T
    bih = p['gru_bih'][None, :]
    bhh = p['gru_bhh'][None, :]
    b0 = p['conv0_b'][None, :]
    b1 = p['conv1_b'][None, :]

    def step(u0, h):
        s0 = _agg(u0.reshape(NC * N, HH), src_all, dst_r)
        u1 = _mid_call(s0, dinv2, b0, w1t)
        s1 = _agg(u1.reshape(NC * N, HH), src_all, dst_r)
        return _gru_call(s1, dinv2, b1, wiht, whht, bih, bhh, h)

    h = jnp.zeros((N, HID), jnp.float32)
    for t in range(T_IN):
        h = step(U0_all[t], h)

    pred_args = (p['pred_W1'].T, p['pred_b1'][None, :], p['pred_ln1_g'][None, :],
                 p['pred_ln1_b'][None, :], p['pred_W2'].T, p['pred_b2'][None, :],
                 p['pred_ln2_g'][None, :], p['pred_ln2_b'][None, :],
                 p['pred_W3'], p['pred_b3'][None, :], linw, linb, w0t)

    outs = []
    u0 = U0_all[T_IN - 1]
    for k in range(T_OUT):
        h = step(u0, h)
        y, u0 = _pred_call(h, *pred_args, dinv2)
        outs.append(y.reshape(NG, 10)[:, 1:].reshape(-1, 1))
    return jnp.concatenate(outs, axis=1)


# speedup vs baseline: 8.5301x; 1.4307x over previous
import functools

import jax
import jax.numpy as jnp
from jax import lax
from jax.experimental import pallas as pl
from jax.experimental.pallas import tpu as pltpu
from jax.experimental.pallas import tpu_sc as plsc

N = 10000
E = 320000
HID = 128
NCG = 4
HG = HID // NCG
NGC = NCG // 2
T_IN = 8
T_OUT = 4
ALPHA = 0.5
D_DEPTH = 2
NG = N // 10

NC = 2
NS = 16
CH = 128
NBUF = 4
K_CH = -(-E // (NS * CH * NBUF)) * NBUF
E_PAD = NS * CH * K_CH
N_PAD = 10240
SLAB = 632
SLAB_LAST = N - 15 * SLAB


def _agg_body(u4_hbm, srcr_hbm, dstr_hbm, out_hbm, src_v, dst_v, rows_v,
              tab_sh, acc_sh, *sems):
    gsems = sems[:NBUF]
    ssems = sems[NBUF:]
    c = lax.axis_index("c")
    s = lax.axis_index("s")
    pltpu.sync_copy(srcr_hbm.at[s], src_v)
    pltpu.sync_copy(dstr_hbm.at[s], dst_v)

    for g in range(NGC):
        cg = c * NGC + g
        @pl.when(s < NS - 1)
        def _():
            pltpu.sync_copy(u4_hbm.at[cg, pl.ds(s * SLAB, SLAB)],
                            tab_sh.at[pl.ds(s * SLAB, SLAB)])
            pltpu.sync_copy(u4_hbm.at[cg, pl.ds(s * SLAB, SLAB)],
                            acc_sh.at[pl.ds(s * SLAB, SLAB)])

        @pl.when(s == NS - 1)
        def _():
            pltpu.sync_copy(u4_hbm.at[cg, pl.ds(15 * SLAB, SLAB_LAST)],
                            tab_sh.at[pl.ds(15 * SLAB, SLAB_LAST)])
            pltpu.sync_copy(u4_hbm.at[cg, pl.ds(15 * SLAB, SLAB_LAST)],
                            acc_sh.at[pl.ds(15 * SLAB, SLAB_LAST)])

        plsc.subcore_barrier()

        for b in range(NBUF):
            pltpu.async_copy(tab_sh.at[src_v.at[b]], rows_v.at[b], gsems[b])

        def group(gi, carry):
            base = gi * NBUF
            for b in range(NBUF):
                j = base + b
                pltpu.make_async_copy(tab_sh.at[src_v.at[j]], rows_v.at[b],
                                      gsems[b]).wait()
                pltpu.async_copy(rows_v.at[b], acc_sh.at[dst_v.at[j]],
                                 ssems[b], add=True)
            for b in range(NBUF):
                jn = base + NBUF + b

                @pl.when(jn < K_CH)
                def _():
                    pltpu.make_async_copy(rows_v.at[b],
                                          acc_sh.at[dst_v.at[base + b]],
                                          ssems[b]).wait()
                    pltpu.async_copy(tab_sh.at[src_v.at[jn]], rows_v.at[b],
                                     gsems[b])
            return carry

        lax.fori_loop(0, K_CH // NBUF, group, 0)
        for b in range(NBUF):
            pltpu.make_async_copy(rows_v.at[b],
                                  acc_sh.at[dst_v.at[K_CH - NBUF + b]],
                                  ssems[b]).wait()
        plsc.subcore_barrier()

        @pl.when(s < NS - 1)
        def _():
            pltpu.sync_copy(acc_sh.at[pl.ds(s * SLAB, SLAB)],
                            out_hbm.at[cg, pl.ds(s * SLAB, SLAB)])

        @pl.when(s == NS - 1)
        def _():
            pltpu.sync_copy(acc_sh.at[pl.ds(15 * SLAB, SLAB_LAST)],
                            out_hbm.at[cg, pl.ds(15 * SLAB, SLAB_LAST)])

        plsc.subcore_barrier()


_agg = pl.kernel(
    _agg_body,
    out_type=jax.ShapeDtypeStruct((NCG, N, HG), jnp.float32),
    mesh=plsc.VectorSubcoreMesh(core_axis_name="c", subcore_axis_name="s",
                                num_cores=NC, num_subcores=NS),
    scratch_types=[
        pltpu.VMEM((K_CH, CH), jnp.int32),
        pltpu.VMEM((K_CH, CH), jnp.int32),
        pltpu.VMEM((NBUF, CH, HG), jnp.float32),
        pltpu.VMEM_SHARED((N, HG), jnp.float32),
        pltpu.VMEM_SHARED((N_PAD, HG), jnp.float32),
    ] + [pltpu.SemaphoreType.DMA] * (2 * NBUF),
    compiler_params=pltpu.CompilerParams(use_tc_tiling_on_sc=False),
)


RB = 2000
NB = N // RB


def _bcast_spec(shape):
    return pl.BlockSpec(shape, lambda i: (0,) * len(shape))


def _row_spec(cols):
    return pl.BlockSpec((RB, cols), lambda i: (i, 0))


def _split_spec():
    return pl.BlockSpec((NCG, RB, HG), lambda i: (0, i, 0))


def _split(u):
    return jnp.stack([u[:, g * HG:(g + 1) * HG] for g in range(NCG)], axis=0)


def _unsplit(ref):
    return jnp.concatenate([ref[g] for g in range(NCG)], axis=1)


def _ghost_kernel(xin_ref, w1t, b1, w2t, g1t, gb1, g2t, out_ref):
    xin = xin_ref[...]
    z = jnp.maximum(jnp.dot(xin, w1t[...]) + b1[...], 0.0)
    delta = jnp.dot(z, w2t[...])
    g = jnp.maximum(jnp.dot(xin, g1t[...]) + gb1[...], 0.0)
    gate = jax.nn.sigmoid(jnp.dot(g, g2t[...]))
    out_ref[...] = delta * gate


def _embed_kernel(x_ref, dinv_ref, linw, linb, w0t, out_ref):
    h0 = jnp.maximum(x_ref[...] * linw[...] + linb[...], 0.0)
    out_ref[...] = _split(jnp.dot(dinv_ref[...] * h0, w0t[...]))[None]


def _mid_kernel(s0_ref, dinv_ref, b0, w1t, out_ref):
    sfull = _unsplit(s0_ref)
    h = jnp.maximum(dinv_ref[...] * sfull + b0[...], 0.0)
    out_ref[...] = _split(jnp.dot(dinv_ref[...] * h, w1t[...]))


def _gru_kernel(s1_ref, dinv_ref, b1, wiht, whht, bih, bhh, h_ref, out_ref):
    sfull = _unsplit(s1_ref)
    g = jnp.maximum(dinv_ref[...] * sfull + b1[...], 0.0)
    gi = jnp.dot(g, wiht[...]) + bih[...]
    gh = jnp.dot(h_ref[...], whht[...]) + bhh[...]
    ir, iz, inn = gi[:, :HID], gi[:, HID:2 * HID], gi[:, 2 * HID:]
    hr, hz, hn = gh[:, :HID], gh[:, HID:2 * HID], gh[:, 2 * HID:]
    r = jax.nn.sigmoid(ir + hr)
    z = jax.nn.sigmoid(iz + hz)
    n = jnp.tanh(inn + r * hn)
    out_ref[...] = (1.0 - z) * n + z * h_ref[...]


def _ln(x, g, b):
    m = jnp.mean(x, axis=-1, keepdims=True)
    v = jnp.mean((x - m) ** 2, axis=-1, keepdims=True)
    return (x - m) / jnp.sqrt(v + 1e-05) * g + b


def _pred_kernel(h_ref, w1t, pb1, g1, bb1, w2t, pb2, g2, bb2, p3, pb3,
                 linw, linb, w0t, dinv_ref, y_ref, u0_ref):
    z1 = jnp.maximum(_ln(jnp.dot(h_ref[...], w1t[...]) + pb1[...],
                         g1[...], bb1[...]), 0.0)
    z2 = jnp.maximum(_ln(jnp.dot(z1, w2t[...]) + pb2[...],
                         g2[...], bb2[...]), 0.0)
    y = jnp.sum(z2 * p3[...], axis=1, keepdims=True) + pb3[...]
    y_ref[...] = y
    h0 = jnp.maximum(y * linw[...] + linb[...], 0.0)
    u0_ref[...] = _split(jnp.dot(dinv_ref[...] * h0, w0t[...]))


_ghost_call = pl.pallas_call(
    _ghost_kernel,
    out_shape=jax.ShapeDtypeStruct((T_IN * NG, 128), jnp.float32),
    in_specs=[_bcast_spec((T_IN * NG, 128)), _bcast_spec((128, 128)),
              _bcast_spec((1, 128)), _bcast_spec((128, 128)),
              _bcast_spec((128, 128)), _bcast_spec((1, 128)),
              _bcast_spec((128, 128))],
    out_specs=_bcast_spec((T_IN * NG, 128)),
    grid=(1,),
)

_embed_call = pl.pallas_call(
    _embed_kernel,
    out_shape=jax.ShapeDtypeStruct((T_IN, NCG, N, HG), jnp.float32),
    in_specs=[pl.BlockSpec((RB, 1), lambda t, i: (t * NB + i, 0)),
              pl.BlockSpec((RB, 1), lambda t, i: (i, 0)),
              pl.BlockSpec((1, HID), lambda t, i: (0, 0)),
              pl.BlockSpec((1, HID), lambda t, i: (0, 0)),
              pl.BlockSpec((HID, HID), lambda t, i: (0, 0))],
    out_specs=pl.BlockSpec((1, NCG, RB, HG), lambda t, i: (t, 0, i, 0)),
    grid=(T_IN, NB),
)

_mid_call = pl.pallas_call(
    _mid_kernel,
    out_shape=jax.ShapeDtypeStruct((NCG, N, HG), jnp.float32),
    in_specs=[_split_spec(), _row_spec(1),
              _bcast_spec((1, HID)), _bcast_spec((HID, HID))],
    out_specs=_split_spec(),
    grid=(NB,),
)

_gru_call = pl.pallas_call(
    _gru_kernel,
    out_shape=jax.ShapeDtypeStruct((N, HID), jnp.float32),
    in_specs=[_split_spec(), _row_spec(1),
              _bcast_spec((1, HID)), _bcast_spec((HID, 3 * HID)),
              _bcast_spec((HID, 3 * HID)), _bcast_spec((1, 3 * HID)),
              _bcast_spec((1, 3 * HID)), _row_spec(HID)],
    out_specs=_row_spec(HID),
    grid=(NB,),
)

_pred_call = pl.pallas_call(
    _pred_kernel,
    out_shape=(jax.ShapeDtypeStruct((N, 1), jnp.float32),
               jax.ShapeDtypeStruct((NCG, N, HG), jnp.float32)),
    in_specs=[_row_spec(HID),
              _bcast_spec((HID, HID)), _bcast_spec((1, HID)),
              _bcast_spec((1, HID)), _bcast_spec((1, HID)),
              _bcast_spec((HID, HID)), _bcast_spec((1, HID)),
              _bcast_spec((1, HID)), _bcast_spec((1, HID)),
              _bcast_spec((1, HID)), _bcast_spec((1, 1)),
              _bcast_spec((1, HID)), _bcast_spec((1, HID)),
              _bcast_spec((HID, HID)), _row_spec(1)],
    out_specs=(_row_spec(1), _split_spec()),
    grid=(NB,),
)


def _pad_w(w, shape):
    out = jnp.zeros(shape, jnp.float32)
    return out.at[:w.shape[0], :w.shape[1]].set(w)


def kernel(x, edge_index, edge_attr, mask, params):
    p = params
    src, dst = edge_index[0], edge_index[1]

    deg = 1.0 + jnp.zeros((N,), jnp.float32).at[dst].add(1.0)
    dinv = 1.0 / jnp.sqrt(deg)
    dinv2 = dinv[:, None]

    pos = jnp.arange(E, dtype=src.dtype)
    pos_min = jnp.full((N,), E, dtype=src.dtype).at[src].min(pos)
    has_first = pos_min < E
    first_pos = jnp.minimum(pos_min, E - 1)
    first_out = jnp.where(has_first, dst[first_pos], jnp.array(-1, src.dtype))
    ghost_idx = jnp.arange(0, N, 10)
    dK = []
    cur = first_out[ghost_idx]
    for _ in range(D_DEPTH):
        nxt = first_out[cur]
        nxt = jnp.where(nxt < 0, cur, nxt)
        dK.append(nxt)
        cur = nxt
    dist = jnp.maximum(edge_attr[:, 0], 1e-06)
    first_dx = jnp.where(has_first, dist[first_pos], jnp.float32(-1.0))
    dx_bnd = jnp.maximum(first_dx[ghost_idx], 1e-06)

    pad = E_PAD - E
    src_r = jnp.concatenate([src, jnp.zeros((pad,), src.dtype)]).reshape(NS, K_CH, CH)
    dst_r = jnp.concatenate([dst, jnp.full((pad,), N, dst.dtype)]).reshape(NS, K_CH, CH)

    X = x[:, :, 0]
    xg = X[0::10]
    d1 = X[dK[0]]
    d2 = X[dK[1]]
    xin = jnp.stack([xg.T, d1.T, d2.T,
                     jnp.broadcast_to(dx_bnd[None, :], (T_IN, NG))], axis=-1)
    xin = xin.reshape(T_IN * NG, 4)
    xin_p = jnp.concatenate([xin, jnp.zeros((T_IN * NG, 124), jnp.float32)], axis=1)
    gf_out = _ghost_call(
        xin_p,
        _pad_w(p['gf_fc1_W'], (128, 128)).T,
        _pad_w(p['gf_fc1_b'][None, :], (1, 128)),
        _pad_w(p['gf_fc2_W'], (128, 128)).T,
        _pad_w(p['gf_g1_W'], (128, 128)).T,
        _pad_w(p['gf_g1_b'][None, :], (1, 128)),
        _pad_w(p['gf_g2_W'], (128, 128)).T,
    )
    dcol = gf_out[:, 0].reshape(T_IN, NG)
    fused = xg.T + ALPHA * dcol
    Xr = X.reshape(NG, 10, T_IN)
    Xf = jnp.concatenate([fused.T[:, None, :], Xr[:, 1:, :]], axis=1).reshape(N, T_IN)

    linw = p['lin_W'][:, 0][None, :]
    linb = p['lin_b'][None, :]
    w0t = p['conv0_W'].T
    w1t = p['conv1_W'].T
    U0_all = _embed_call(Xf.T.reshape(T_IN * N, 1), dinv2, linw, linb, w0t)

    wiht = p['gru_Wih'].T
    whht = p['gru_Whh'].T
    bih = p['gru_bih'][None, :]
    bhh = p['gru_bhh'][None, :]
    b0 = p['conv0_b'][None, :]
    b1 = p['conv1_b'][None, :]

    def step(u0, h):
        s0 = _agg(u0, src_r, dst_r)
        u1 = _mid_call(s0, dinv2, b0, w1t)
        s1 = _agg(u1, src_r, dst_r)
        return _gru_call(s1, dinv2, b1, wiht, whht, bih, bhh, h)

    h = jnp.zeros((N, HID), jnp.float32)
    for t in range(T_IN):
        h = step(U0_all[t], h)

    pred_args = (p['pred_W1'].T, p['pred_b1'][None, :], p['pred_ln1_g'][None, :],
                 p['pred_ln1_b'][None, :], p['pred_W2'].T, p['pred_b2'][None, :],
                 p['pred_ln2_g'][None, :], p['pred_ln2_b'][None, :],
                 p['pred_W3'], p['pred_b3'][None, :], linw, linb, w0t)

    outs = []
    u0 = U0_all[T_IN - 1]
    for k in range(T_OUT):
        h = step(u0, h)
        y, u0 = _pred_call(h, *pred_args, dinv2)
        outs.append(y.reshape(NG, 10)[:, 1:].reshape(-1, 1))
    return jnp.concatenate(outs, axis=1)


# speedup vs baseline: 9.9547x; 1.1670x over previous
import functools

import jax
import jax.numpy as jnp
from jax import lax
from jax.experimental import pallas as pl
from jax.experimental.pallas import tpu as pltpu
from jax.experimental.pallas import tpu_sc as plsc

N = 10000
E = 320000
HID = 128
NCG = 4
HG = HID // NCG
NGC = NCG // 2
T_IN = 8
T_OUT = 4
ALPHA = 0.5
D_DEPTH = 2
NG = N // 10

NC = 2
NS = 16
CH = 128
NBUF = 4
K_CH = -(-E // (NS * CH * 2 * NBUF)) * 2 * NBUF
E_PAD = NS * CH * K_CH
N_PAD = 10240
SLAB = 632
SLAB_LAST = N - 15 * SLAB


def _agg_body(u4_hbm, srcr_hbm, dstr_hbm, out_hbm, src_v, dst_v, rows_v,
              tab_sh, acc_sh, *sems):
    gsems = sems[:2]
    ssems = sems[2:]
    c = lax.axis_index("c")
    s = lax.axis_index("s")
    pltpu.sync_copy(srcr_hbm.at[s], src_v)
    pltpu.sync_copy(dstr_hbm.at[s], dst_v)

    for g in range(NGC):
        cg = c * NGC + g
        @pl.when(s < NS - 1)
        def _():
            pltpu.sync_copy(u4_hbm.at[cg, pl.ds(s * SLAB, SLAB)],
                            tab_sh.at[pl.ds(s * SLAB, SLAB)])
            pltpu.sync_copy(u4_hbm.at[cg, pl.ds(s * SLAB, SLAB)],
                            acc_sh.at[pl.ds(s * SLAB, SLAB)])

        @pl.when(s == NS - 1)
        def _():
            pltpu.sync_copy(u4_hbm.at[cg, pl.ds(15 * SLAB, SLAB_LAST)],
                            tab_sh.at[pl.ds(15 * SLAB, SLAB_LAST)])
            pltpu.sync_copy(u4_hbm.at[cg, pl.ds(15 * SLAB, SLAB_LAST)],
                            acc_sh.at[pl.ds(15 * SLAB, SLAB_LAST)])

        plsc.subcore_barrier()

        def fire_gathers(h, base):
            for q in range(NBUF):
                pltpu.async_copy(tab_sh.at[src_v.at[base + q]],
                                 rows_v.at[h, q], gsems[h])

        def fire_scatters(h, base):
            for q in range(NBUF):
                pltpu.async_copy(rows_v.at[h, q],
                                 acc_sh.at[dst_v.at[base + q]],
                                 ssems[h], add=True)

        def wait_half(sem):
            pltpu.make_async_copy(u4_hbm.at[:, pl.ds(0, CH)],
                                  rows_v.at[0], sem).wait()

        fire_gathers(0, 0)

        def pair(pi, carry):
            base = pi * 2 * NBUF
            wait_half(gsems[0])
            fire_scatters(0, base)

            @pl.when(pi > 0)
            def _():
                wait_half(ssems[1])

            fire_gathers(1, base + NBUF)
            wait_half(gsems[1])
            fire_scatters(1, base + NBUF)
            wait_half(ssems[0])

            @pl.when(base + 2 * NBUF < K_CH)
            def _():
                fire_gathers(0, base + 2 * NBUF)
            return carry

        lax.fori_loop(0, K_CH // (2 * NBUF), pair, 0)
        wait_half(ssems[1])
        plsc.subcore_barrier()

        @pl.when(s < NS - 1)
        def _():
            pltpu.sync_copy(acc_sh.at[pl.ds(s * SLAB, SLAB)],
                            out_hbm.at[cg, pl.ds(s * SLAB, SLAB)])

        @pl.when(s == NS - 1)
        def _():
            pltpu.sync_copy(acc_sh.at[pl.ds(15 * SLAB, SLAB_LAST)],
                            out_hbm.at[cg, pl.ds(15 * SLAB, SLAB_LAST)])

        plsc.subcore_barrier()


_agg = pl.kernel(
    _agg_body,
    out_type=jax.ShapeDtypeStruct((NCG, N, HG), jnp.float32),
    mesh=plsc.VectorSubcoreMesh(core_axis_name="c", subcore_axis_name="s",
                                num_cores=NC, num_subcores=NS),
    scratch_types=[
        pltpu.VMEM((K_CH, CH), jnp.int32),
        pltpu.VMEM((K_CH, CH), jnp.int32),
        pltpu.VMEM((2, NBUF, CH, HG), jnp.float32),
        pltpu.VMEM_SHARED((N, HG), jnp.float32),
        pltpu.VMEM_SHARED((N_PAD, HG), jnp.float32),
    ] + [pltpu.SemaphoreType.DMA] * 4,
    compiler_params=pltpu.CompilerParams(use_tc_tiling_on_sc=False),
)


RB = 2000
NB = N // RB


def _bcast_spec(shape):
    return pl.BlockSpec(shape, lambda i: (0,) * len(shape))


def _row_spec(cols):
    return pl.BlockSpec((RB, cols), lambda i: (i, 0))


def _split_spec():
    return pl.BlockSpec((NCG, RB, HG), lambda i: (0, i, 0))


def _split(u):
    return jnp.stack([u[:, g * HG:(g + 1) * HG] for g in range(NCG)], axis=0)


def _unsplit(ref):
    return jnp.concatenate([ref[g] for g in range(NCG)], axis=1)


def _ghost_kernel(xin_ref, w1t, b1, w2t, g1t, gb1, g2t, out_ref):
    xin = xin_ref[...]
    z = jnp.maximum(jnp.dot(xin, w1t[...]) + b1[...], 0.0)
    delta = jnp.dot(z, w2t[...])
    g = jnp.maximum(jnp.dot(xin, g1t[...]) + gb1[...], 0.0)
    gate = jax.nn.sigmoid(jnp.dot(g, g2t[...]))
    out_ref[...] = delta * gate


def _embed_kernel(x_ref, dinv_ref, linw, linb, w0t, out_ref):
    h0 = jnp.maximum(x_ref[...] * linw[...] + linb[...], 0.0)
    out_ref[...] = _split(jnp.dot(dinv_ref[...] * h0, w0t[...]))[None]


def _mid_kernel(s0_ref, dinv_ref, b0, w1t, out_ref):
    sfull = _unsplit(s0_ref)
    h = jnp.maximum(dinv_ref[...] * sfull + b0[...], 0.0)
    out_ref[...] = _split(jnp.dot(dinv_ref[...] * h, w1t[...]))


def _gru_kernel(s1_ref, dinv_ref, b1, wiht, whht, bih, bhh, h_ref, out_ref):
    sfull = _unsplit(s1_ref)
    g = jnp.maximum(dinv_ref[...] * sfull + b1[...], 0.0)
    gi = jnp.dot(g, wiht[...]) + bih[...]
    gh = jnp.dot(h_ref[...], whht[...]) + bhh[...]
    ir, iz, inn = gi[:, :HID], gi[:, HID:2 * HID], gi[:, 2 * HID:]
    hr, hz, hn = gh[:, :HID], gh[:, HID:2 * HID], gh[:, 2 * HID:]
    r = jax.nn.sigmoid(ir + hr)
    z = jax.nn.sigmoid(iz + hz)
    n = jnp.tanh(inn + r * hn)
    out_ref[...] = (1.0 - z) * n + z * h_ref[...]


def _ln(x, g, b):
    m = jnp.mean(x, axis=-1, keepdims=True)
    v = jnp.mean((x - m) ** 2, axis=-1, keepdims=True)
    return (x - m) / jnp.sqrt(v + 1e-05) * g + b


def _pred_kernel(h_ref, w1t, pb1, g1, bb1, w2t, pb2, g2, bb2, p3, pb3,
                 linw, linb, w0t, dinv_ref, y_ref, u0_ref):
    z1 = jnp.maximum(_ln(jnp.dot(h_ref[...], w1t[...]) + pb1[...],
                         g1[...], bb1[...]), 0.0)
    z2 = jnp.maximum(_ln(jnp.dot(z1, w2t[...]) + pb2[...],
                         g2[...], bb2[...]), 0.0)
    y = jnp.sum(z2 * p3[...], axis=1, keepdims=True) + pb3[...]
    y_ref[...] = y
    h0 = jnp.maximum(y * linw[...] + linb[...], 0.0)
    u0_ref[...] = _split(jnp.dot(dinv_ref[...] * h0, w0t[...]))


_ghost_call = pl.pallas_call(
    _ghost_kernel,
    out_shape=jax.ShapeDtypeStruct((T_IN * NG, 128), jnp.float32),
    in_specs=[_bcast_spec((T_IN * NG, 128)), _bcast_spec((128, 128)),
              _bcast_spec((1, 128)), _bcast_spec((128, 128)),
              _bcast_spec((128, 128)), _bcast_spec((1, 128)),
              _bcast_spec((128, 128))],
    out_specs=_bcast_spec((T_IN * NG, 128)),
    grid=(1,),
)

_embed_call = pl.pallas_call(
    _embed_kernel,
    out_shape=jax.ShapeDtypeStruct((T_IN, NCG, N, HG), jnp.float32),
    in_specs=[pl.BlockSpec((RB, 1), lambda t, i: (t * NB + i, 0)),
              pl.BlockSpec((RB, 1), lambda t, i: (i, 0)),
              pl.BlockSpec((1, HID), lambda t, i: (0, 0)),
              pl.BlockSpec((1, HID), lambda t, i: (0, 0)),
              pl.BlockSpec((HID, HID), lambda t, i: (0, 0))],
    out_specs=pl.BlockSpec((1, NCG, RB, HG), lambda t, i: (t, 0, i, 0)),
    grid=(T_IN, NB),
)

_mid_call = pl.pallas_call(
    _mid_kernel,
    out_shape=jax.ShapeDtypeStruct((NCG, N, HG), jnp.float32),
    in_specs=[_split_spec(), _row_spec(1),
              _bcast_spec((1, HID)), _bcast_spec((HID, HID))],
    out_specs=_split_spec(),
    grid=(NB,),
)

_gru_call = pl.pallas_call(
    _gru_kernel,
    out_shape=jax.ShapeDtypeStruct((N, HID), jnp.float32),
    in_specs=[_split_spec(), _row_spec(1),
              _bcast_spec((1, HID)), _bcast_spec((HID, 3 * HID)),
              _bcast_spec((HID, 3 * HID)), _bcast_spec((1, 3 * HID)),
              _bcast_spec((1, 3 * HID)), _row_spec(HID)],
    out_specs=_row_spec(HID),
    grid=(NB,),
)

_pred_call = pl.pallas_call(
    _pred_kernel,
    out_shape=(jax.ShapeDtypeStruct((N, 1), jnp.float32),
               jax.ShapeDtypeStruct((NCG, N, HG), jnp.float32)),
    in_specs=[_row_spec(HID),
              _bcast_spec((HID, HID)), _bcast_spec((1, HID)),
              _bcast_spec((1, HID)), _bcast_spec((1, HID)),
              _bcast_spec((HID, HID)), _bcast_spec((1, HID)),
              _bcast_spec((1, HID)), _bcast_spec((1, HID)),
              _bcast_spec((1, HID)), _bcast_spec((1, 1)),
              _bcast_spec((1, HID)), _bcast_spec((1, HID)),
              _bcast_spec((HID, HID)), _row_spec(1)],
    out_specs=(_row_spec(1), _split_spec()),
    grid=(NB,),
)


def _pad_w(w, shape):
    out = jnp.zeros(shape, jnp.float32)
    return out.at[:w.shape[0], :w.shape[1]].set(w)


def kernel(x, edge_index, edge_attr, mask, params):
    p = params
    src, dst = edge_index[0], edge_index[1]

    deg = 1.0 + jnp.zeros((N,), jnp.float32).at[dst].add(1.0)
    dinv = 1.0 / jnp.sqrt(deg)
    dinv2 = dinv[:, None]

    pos = jnp.arange(E, dtype=src.dtype)
    pos_min = jnp.full((N,), E, dtype=src.dtype).at[src].min(pos)
    has_first = pos_min < E
    first_pos = jnp.minimum(pos_min, E - 1)
    first_out = jnp.where(has_first, dst[first_pos], jnp.array(-1, src.dtype))
    ghost_idx = jnp.arange(0, N, 10)
    dK = []
    cur = first_out[ghost_idx]
    for _ in range(D_DEPTH):
        nxt = first_out[cur]
        nxt = jnp.where(nxt < 0, cur, nxt)
        dK.append(nxt)
        cur = nxt
    dist = jnp.maximum(edge_attr[:, 0], 1e-06)
    first_dx = jnp.where(has_first, dist[first_pos], jnp.float32(-1.0))
    dx_bnd = jnp.maximum(first_dx[ghost_idx], 1e-06)

    pad = E_PAD - E
    src_r = jnp.concatenate([src, jnp.zeros((pad,), src.dtype)]).reshape(NS, K_CH, CH)
    dst_r = jnp.concatenate([dst, jnp.full((pad,), N, dst.dtype)]).reshape(NS, K_CH, CH)

    X = x[:, :, 0]
    xg = X[0::10]
    d1 = X[dK[0]]
    d2 = X[dK[1]]
    xin = jnp.stack([xg.T, d1.T, d2.T,
                     jnp.broadcast_to(dx_bnd[None, :], (T_IN, NG))], axis=-1)
    xin = xin.reshape(T_IN * NG, 4)
    xin_p = jnp.concatenate([xin, jnp.zeros((T_IN * NG, 124), jnp.float32)], axis=1)
    gf_out = _ghost_call(
        xin_p,
        _pad_w(p['gf_fc1_W'], (128, 128)).T,
        _pad_w(p['gf_fc1_b'][None, :], (1, 128)),
        _pad_w(p['gf_fc2_W'], (128, 128)).T,
        _pad_w(p['gf_g1_W'], (128, 128)).T,
        _pad_w(p['gf_g1_b'][None, :], (1, 128)),
        _pad_w(p['gf_g2_W'], (128, 128)).T,
    )
    dcol = gf_out[:, 0].reshape(T_IN, NG)
    fused = xg.T + ALPHA * dcol
    Xr = X.reshape(NG, 10, T_IN)
    Xf = jnp.concatenate([fused.T[:, None, :], Xr[:, 1:, :]], axis=1).reshape(N, T_IN)

    linw = p['lin_W'][:, 0][None, :]
    linb = p['lin_b'][None, :]
    w0t = p['conv0_W'].T
    w1t = p['conv1_W'].T
    U0_all = _embed_call(Xf.T.reshape(T_IN * N, 1), dinv2, linw, linb, w0t)

    wiht = p['gru_Wih'].T
    whht = p['gru_Whh'].T
    bih = p['gru_bih'][None, :]
    bhh = p['gru_bhh'][None, :]
    b0 = p['conv0_b'][None, :]
    b1 = p['conv1_b'][None, :]

    def step(u0, h):
        s0 = _agg(u0, src_r, dst_r)
        u1 = _mid_call(s0, dinv2, b0, w1t)
        s1 = _agg(u1, src_r, dst_r)
        return _gru_call(s1, dinv2, b1, wiht, whht, bih, bhh, h)

    h = jnp.zeros((N, HID), jnp.float32)
    for t in range(T_IN):
        h = step(U0_all[t], h)

    pred_args = (p['pred_W1'].T, p['pred_b1'][None, :], p['pred_ln1_g'][None, :],
                 p['pred_ln1_b'][None, :], p['pred_W2'].T, p['pred_b2'][None, :],
                 p['pred_ln2_g'][None, :], p['pred_ln2_b'][None, :],
                 p['pred_W3'], p['pred_b3'][None, :], linw, linb, w0t)

    outs = []
    u0 = U0_all[T_IN - 1]
    for k in range(T_OUT):
        h = step(u0, h)
        y, u0 = _pred_call(h, *pred_args, dinv2)
        outs.append(y.reshape(NG, 10)[:, 1:].reshape(-1, 1))
    return jnp.concatenate(outs, axis=1)


# speedup vs baseline: 10.7876x; 1.0837x over previous
import functools

import jax
import jax.numpy as jnp
from jax import lax
from jax.experimental import pallas as pl
from jax.experimental.pallas import tpu as pltpu
from jax.experimental.pallas import tpu_sc as plsc

N = 10000
E = 320000
HID = 128
NCG = 4
HG = HID // NCG
NGC = NCG // 2
T_IN = 8
T_OUT = 4
ALPHA = 0.5
D_DEPTH = 2
NG = N // 10

NC = 2
NS = 16
CH = 128
NBUF = 4
K_CH = -(-E // (NS * CH * 2 * NBUF)) * 2 * NBUF
E_PAD = NS * CH * K_CH
N_PAD = 10240
SLAB = 632
SLAB_LAST = N - 15 * SLAB


def _agg_body(u4_hbm, srcr_hbm, dstr_hbm, out_hbm, src_v, dst_v, rows_v,
              tab_sh, acc_sh, *sems):
    gsems = sems[:2]
    ssems = sems[2:]
    c = lax.axis_index("c")
    s = lax.axis_index("s")
    pltpu.sync_copy(srcr_hbm.at[s], src_v)
    pltpu.sync_copy(dstr_hbm.at[s], dst_v)

    for g in range(NGC):
        cg = c * NGC + g
        @pl.when(s < NS - 1)
        def _():
            pltpu.sync_copy(u4_hbm.at[cg, pl.ds(s * SLAB, SLAB)],
                            tab_sh.at[pl.ds(s * SLAB, SLAB)])
            pltpu.sync_copy(u4_hbm.at[cg, pl.ds(s * SLAB, SLAB)],
                            acc_sh.at[pl.ds(s * SLAB, SLAB)])

        @pl.when(s == NS - 1)
        def _():
            pltpu.sync_copy(u4_hbm.at[cg, pl.ds(15 * SLAB, SLAB_LAST)],
                            tab_sh.at[pl.ds(15 * SLAB, SLAB_LAST)])
            pltpu.sync_copy(u4_hbm.at[cg, pl.ds(15 * SLAB, SLAB_LAST)],
                            acc_sh.at[pl.ds(15 * SLAB, SLAB_LAST)])

        plsc.subcore_barrier()

        def fire_gathers(h, base):
            for q in range(NBUF):
                pltpu.async_copy(tab_sh.at[src_v.at[base + q]],
                                 rows_v.at[h, q], gsems[h])

        def fire_scatters(h, base):
            for q in range(NBUF):
                pltpu.async_copy(rows_v.at[h, q],
                                 acc_sh.at[dst_v.at[base + q]],
                                 ssems[h], add=True)

        def wait_half(sem):
            pltpu.make_async_copy(u4_hbm.at[:, pl.ds(0, CH)],
                                  rows_v.at[0], sem).wait()

        fire_gathers(0, 0)

        def pair(pi, carry):
            base = pi * 2 * NBUF
            wait_half(gsems[0])
            fire_scatters(0, base)

            @pl.when(pi > 0)
            def _():
                wait_half(ssems[1])

            fire_gathers(1, base + NBUF)
            wait_half(gsems[1])
            fire_scatters(1, base + NBUF)
            wait_half(ssems[0])

            @pl.when(base + 2 * NBUF < K_CH)
            def _():
                fire_gathers(0, base + 2 * NBUF)
            return carry

        lax.fori_loop(0, K_CH // (2 * NBUF), pair, 0)
        wait_half(ssems[1])
        plsc.subcore_barrier()

        @pl.when(s < NS - 1)
        def _():
            pltpu.sync_copy(acc_sh.at[pl.ds(s * SLAB, SLAB)],
                            out_hbm.at[cg, pl.ds(s * SLAB, SLAB)])

        @pl.when(s == NS - 1)
        def _():
            pltpu.sync_copy(acc_sh.at[pl.ds(15 * SLAB, SLAB_LAST)],
                            out_hbm.at[cg, pl.ds(15 * SLAB, SLAB_LAST)])

        plsc.subcore_barrier()


_agg = pl.kernel(
    _agg_body,
    out_type=jax.ShapeDtypeStruct((NCG, N, HG), jnp.float32),
    mesh=plsc.VectorSubcoreMesh(core_axis_name="c", subcore_axis_name="s",
                                num_cores=NC, num_subcores=NS),
    scratch_types=[
        pltpu.VMEM((K_CH, CH), jnp.int32),
        pltpu.VMEM((K_CH, CH), jnp.int32),
        pltpu.VMEM((2, NBUF, CH, HG), jnp.float32),
        pltpu.VMEM_SHARED((N, HG), jnp.float32),
        pltpu.VMEM_SHARED((N_PAD, HG), jnp.float32),
    ] + [pltpu.SemaphoreType.DMA] * 4,
    compiler_params=pltpu.CompilerParams(use_tc_tiling_on_sc=False),
)


DR = 640
R_PER = DR // NS


def _deg_body(dstr_hbm, out_hbm, dst_v, tl, buf, red, sh):
    c = lax.axis_index("c")
    s = lax.axis_index("s")

    @pl.when(c == 0)
    def _():
        pltpu.sync_copy(dstr_hbm.at[s], dst_v)
        ones = jnp.ones((16,), jnp.float32)

        def z(i, carry):
            tl[i] = jnp.zeros((16,), jnp.float32)
            return carry

        lax.fori_loop(0, DR, z, 0)

        def body(j, carry):
            for q in range(CH // 16):
                d16 = dst_v[j, pl.ds(q * 16, 16)]
                rows = lax.shift_right_logical(d16, 4)
                cols = lax.bitwise_and(d16, 15)
                plsc.addupdate_scatter(tl, [rows, cols], ones)
            return carry

        lax.fori_loop(0, K_CH, body, 0)
        pltpu.sync_copy(tl, sh.at[s])
        plsc.subcore_barrier()

        def z2(i, carry):
            red[i] = jnp.zeros((16,), jnp.float32)
            return carry

        lax.fori_loop(0, R_PER, z2, 0)

        def rbody(u, carry):
            pltpu.sync_copy(sh.at[u, pl.ds(s * R_PER, R_PER)], buf)

            def add(r, c2):
                red[r] = red[r] + buf[r]
                return c2

            lax.fori_loop(0, R_PER, add, 0)
            return carry

        lax.fori_loop(0, NS, rbody, 0)
        pltpu.sync_copy(red, out_hbm.at[pl.ds(s * R_PER, R_PER)])


_deg = pl.kernel(
    _deg_body,
    out_type=jax.ShapeDtypeStruct((DR, 16), jnp.float32),
    mesh=plsc.VectorSubcoreMesh(core_axis_name="c", subcore_axis_name="s",
                                num_cores=NC, num_subcores=NS),
    scratch_types=[
        pltpu.VMEM((K_CH, CH), jnp.int32),
        pltpu.VMEM((DR, 16), jnp.float32),
        pltpu.VMEM((R_PER, 16), jnp.float32),
        pltpu.VMEM((R_PER, 16), jnp.float32),
        pltpu.VMEM_SHARED((NS, DR, 16), jnp.float32),
    ],
    compiler_params=pltpu.CompilerParams(use_tc_tiling_on_sc=False,
                                         needs_layout_passes=False),
)


RB = 2000
NB = N // RB


def _bcast_spec(shape):
    return pl.BlockSpec(shape, lambda i: (0,) * len(shape))


def _row_spec(cols):
    return pl.BlockSpec((RB, cols), lambda i: (i, 0))


def _split_spec():
    return pl.BlockSpec((NCG, RB, HG), lambda i: (0, i, 0))


def _split(u):
    return jnp.stack([u[:, g * HG:(g + 1) * HG] for g in range(NCG)], axis=0)


def _unsplit(ref):
    return jnp.concatenate([ref[g] for g in range(NCG)], axis=1)


def _ghost_kernel(xin_ref, w1t, b1, w2t, g1t, gb1, g2t, out_ref):
    xin = xin_ref[...]
    z = jnp.maximum(jnp.dot(xin, w1t[...]) + b1[...], 0.0)
    delta = jnp.dot(z, w2t[...])
    g = jnp.maximum(jnp.dot(xin, g1t[...]) + gb1[...], 0.0)
    gate = jax.nn.sigmoid(jnp.dot(g, g2t[...]))
    out_ref[...] = delta * gate


def _embed_kernel(x_ref, dinv_ref, linw, linb, w0t, out_ref):
    h0 = jnp.maximum(x_ref[...] * linw[...] + linb[...], 0.0)
    out_ref[...] = _split(jnp.dot(dinv_ref[...] * h0, w0t[...]))[None]


def _mid_kernel(s0_ref, dinv_ref, b0, w1t, out_ref):
    sfull = _unsplit(s0_ref)
    h = jnp.maximum(dinv_ref[...] * sfull + b0[...], 0.0)
    out_ref[...] = _split(jnp.dot(dinv_ref[...] * h, w1t[...]))


def _gru_kernel(s1_ref, dinv_ref, b1, wiht, whht, bih, bhh, h_ref, out_ref):
    sfull = _unsplit(s1_ref)
    g = jnp.maximum(dinv_ref[...] * sfull + b1[...], 0.0)
    gi = jnp.dot(g, wiht[...]) + bih[...]
    gh = jnp.dot(h_ref[...], whht[...]) + bhh[...]
    ir, iz, inn = gi[:, :HID], gi[:, HID:2 * HID], gi[:, 2 * HID:]
    hr, hz, hn = gh[:, :HID], gh[:, HID:2 * HID], gh[:, 2 * HID:]
    r = jax.nn.sigmoid(ir + hr)
    z = jax.nn.sigmoid(iz + hz)
    n = jnp.tanh(inn + r * hn)
    out_ref[...] = (1.0 - z) * n + z * h_ref[...]


def _ln(x, g, b):
    m = jnp.mean(x, axis=-1, keepdims=True)
    v = jnp.mean((x - m) ** 2, axis=-1, keepdims=True)
    return (x - m) / jnp.sqrt(v + 1e-05) * g + b


def _pred_kernel(h_ref, w1t, pb1, g1, bb1, w2t, pb2, g2, bb2, p3, pb3,
                 linw, linb, w0t, dinv_ref, y_ref, u0_ref):
    z1 = jnp.maximum(_ln(jnp.dot(h_ref[...], w1t[...]) + pb1[...],
                         g1[...], bb1[...]), 0.0)
    z2 = jnp.maximum(_ln(jnp.dot(z1, w2t[...]) + pb2[...],
                         g2[...], bb2[...]), 0.0)
    y = jnp.sum(z2 * p3[...], axis=1, keepdims=True) + pb3[...]
    y_ref[...] = y
    h0 = jnp.maximum(y * linw[...] + linb[...], 0.0)
    u0_ref[...] = _split(jnp.dot(dinv_ref[...] * h0, w0t[...]))


_ghost_call = pl.pallas_call(
    _ghost_kernel,
    out_shape=jax.ShapeDtypeStruct((T_IN * NG, 128), jnp.float32),
    in_specs=[_bcast_spec((T_IN * NG, 128)), _bcast_spec((128, 128)),
              _bcast_spec((1, 128)), _bcast_spec((128, 128)),
              _bcast_spec((128, 128)), _bcast_spec((1, 128)),
              _bcast_spec((128, 128))],
    out_specs=_bcast_spec((T_IN * NG, 128)),
    grid=(1,),
)

_embed_call = pl.pallas_call(
    _embed_kernel,
    out_shape=jax.ShapeDtypeStruct((T_IN, NCG, N, HG), jnp.float32),
    in_specs=[pl.BlockSpec((RB, 1), lambda t, i: (t * NB + i, 0)),
              pl.BlockSpec((RB, 1), lambda t, i: (i, 0)),
              pl.BlockSpec((1, HID), lambda t, i: (0, 0)),
              pl.BlockSpec((1, HID), lambda t, i: (0, 0)),
              pl.BlockSpec((HID, HID), lambda t, i: (0, 0))],
    out_specs=pl.BlockSpec((1, NCG, RB, HG), lambda t, i: (t, 0, i, 0)),
    grid=(T_IN, NB),
)

_mid_call = pl.pallas_call(
    _mid_kernel,
    out_shape=jax.ShapeDtypeStruct((NCG, N, HG), jnp.float32),
    in_specs=[_split_spec(), _row_spec(1),
              _bcast_spec((1, HID)), _bcast_spec((HID, HID))],
    out_specs=_split_spec(),
    grid=(NB,),
)

_gru_call = pl.pallas_call(
    _gru_kernel,
    out_shape=jax.ShapeDtypeStruct((N, HID), jnp.float32),
    in_specs=[_split_spec(), _row_spec(1),
              _bcast_spec((1, HID)), _bcast_spec((HID, 3 * HID)),
              _bcast_spec((HID, 3 * HID)), _bcast_spec((1, 3 * HID)),
              _bcast_spec((1, 3 * HID)), _row_spec(HID)],
    out_specs=_row_spec(HID),
    grid=(NB,),
)

_pred_call = pl.pallas_call(
    _pred_kernel,
    out_shape=(jax.ShapeDtypeStruct((N, 1), jnp.float32),
               jax.ShapeDtypeStruct((NCG, N, HG), jnp.float32)),
    in_specs=[_row_spec(HID),
              _bcast_spec((HID, HID)), _bcast_spec((1, HID)),
              _bcast_spec((1, HID)), _bcast_spec((1, HID)),
              _bcast_spec((HID, HID)), _bcast_spec((1, HID)),
              _bcast_spec((1, HID)), _bcast_spec((1, HID)),
              _bcast_spec((1, HID)), _bcast_spec((1, 1)),
              _bcast_spec((1, HID)), _bcast_spec((1, HID)),
              _bcast_spec((HID, HID)), _row_spec(1)],
    out_specs=(_row_spec(1), _split_spec()),
    grid=(NB,),
)


def _pad_w(w, shape):
    out = jnp.zeros(shape, jnp.float32)
    return out.at[:w.shape[0], :w.shape[1]].set(w)


def kernel(x, edge_index, edge_attr, mask, params):
    p = params
    src, dst = edge_index[0], edge_index[1]

    pad = E_PAD - E
    dst_r = jnp.concatenate([dst, jnp.full((pad,), N, dst.dtype)]).reshape(NS, K_CH, CH)
    deg = 1.0 + _deg(dst_r).reshape(-1)[:N]
    dinv = 1.0 / jnp.sqrt(deg)
    dinv2 = dinv[:, None]

    pos = jnp.arange(E, dtype=src.dtype)
    pos_min = jnp.full((N,), E, dtype=src.dtype).at[src].min(pos)
    has_first = pos_min < E
    first_pos = jnp.minimum(pos_min, E - 1)
    first_out = jnp.where(has_first, dst[first_pos], jnp.array(-1, src.dtype))
    ghost_idx = jnp.arange(0, N, 10)
    dK = []
    cur = first_out[ghost_idx]
    for _ in range(D_DEPTH):
        nxt = first_out[cur]
        nxt = jnp.where(nxt < 0, cur, nxt)
        dK.append(nxt)
        cur = nxt
    dist = jnp.maximum(edge_attr[:, 0], 1e-06)
    first_dx = jnp.where(has_first, dist[first_pos], jnp.float32(-1.0))
    dx_bnd = jnp.maximum(first_dx[ghost_idx], 1e-06)

    src_r = jnp.concatenate([src, jnp.zeros((pad,), src.dtype)]).reshape(NS, K_CH, CH)

    X = x[:, :, 0]
    xg = X[0::10]
    d1 = X[dK[0]]
    d2 = X[dK[1]]
    xin = jnp.stack([xg.T, d1.T, d2.T,
                     jnp.broadcast_to(dx_bnd[None, :], (T_IN, NG))], axis=-1)
    xin = xin.reshape(T_IN * NG, 4)
    xin_p = jnp.concatenate([xin, jnp.zeros((T_IN * NG, 124), jnp.float32)], axis=1)
    gf_out = _ghost_call(
        xin_p,
        _pad_w(p['gf_fc1_W'], (128, 128)).T,
        _pad_w(p['gf_fc1_b'][None, :], (1, 128)),
        _pad_w(p['gf_fc2_W'], (128, 128)).T,
        _pad_w(p['gf_g1_W'], (128, 128)).T,
        _pad_w(p['gf_g1_b'][None, :], (1, 128)),
        _pad_w(p['gf_g2_W'], (128, 128)).T,
    )
    dcol = gf_out[:, 0].reshape(T_IN, NG)
    fused = xg.T + ALPHA * dcol
    Xr = X.reshape(NG, 10, T_IN)
    Xf = jnp.concatenate([fused.T[:, None, :], Xr[:, 1:, :]], axis=1).reshape(N, T_IN)

    linw = p['lin_W'][:, 0][None, :]
    linb = p['lin_b'][None, :]
    w0t = p['conv0_W'].T
    w1t = p['conv1_W'].T
    U0_all = _embed_call(Xf.T.reshape(T_IN * N, 1), dinv2, linw, linb, w0t)

    wiht = p['gru_Wih'].T
    whht = p['gru_Whh'].T
    bih = p['gru_bih'][None, :]
    bhh = p['gru_bhh'][None, :]
    b0 = p['conv0_b'][None, :]
    b1 = p['conv1_b'][None, :]

    def step(u0, h):
        s0 = _agg(u0, src_r, dst_r)
        u1 = _mid_call(s0, dinv2, b0, w1t)
        s1 = _agg(u1, src_r, dst_r)
        return _gru_call(s1, dinv2, b1, wiht, whht, bih, bhh, h)

    h = jnp.zeros((N, HID), jnp.float32)
    for t in range(T_IN):
        h = step(U0_all[t], h)

    pred_args = (p['pred_W1'].T, p['pred_b1'][None, :], p['pred_ln1_g'][None, :],
                 p['pred_ln1_b'][None, :], p['pred_W2'].T, p['pred_b2'][None, :],
                 p['pred_ln2_g'][None, :], p['pred_ln2_b'][None, :],
                 p['pred_W3'], p['pred_b3'][None, :], linw, linb, w0t)

    outs = []
    u0 = U0_all[T_IN - 1]
    for k in range(T_OUT):
        h = step(u0, h)
        y, u0 = _pred_call(h, *pred_args, dinv2)
        outs.append(y.reshape(NG, 10)[:, 1:].reshape(-1, 1))
    return jnp.concatenate(outs, axis=1)


# speedup vs baseline: 11.6159x; 1.0768x over previous
import functools

import jax
import jax.numpy as jnp
from jax import lax
from jax.experimental import pallas as pl
from jax.experimental.pallas import tpu as pltpu
from jax.experimental.pallas import tpu_sc as plsc

N = 10000
E = 320000
HID = 128
NCG = 4
HG = HID // NCG
NGC = NCG // 2
T_IN = 8
T_OUT = 4
ALPHA = 0.5
D_DEPTH = 2
NG = N // 10

NC = 2
NS = 16
CH = 128
NBUF = 4
K_CH = -(-E // (NS * CH * 2 * NBUF)) * 2 * NBUF
E_PAD = NS * CH * K_CH
N_PAD = 10240
SLAB = 632
SLAB_LAST = N - 15 * SLAB


def _agg_body(u4_hbm, srcr_hbm, dstr_hbm, out_hbm, src_v, dst_v, rows_v,
              tab_sh, acc_sh, *sems):
    gsems = sems[:2]
    ssems = sems[2:]
    c = lax.axis_index("c")
    s = lax.axis_index("s")
    pltpu.sync_copy(srcr_hbm.at[s], src_v)
    pltpu.sync_copy(dstr_hbm.at[s], dst_v)

    for g in range(NGC):
        cg = c * NGC + g
        @pl.when(s < NS - 1)
        def _():
            pltpu.sync_copy(u4_hbm.at[cg, pl.ds(s * SLAB, SLAB)],
                            tab_sh.at[pl.ds(s * SLAB, SLAB)])
            pltpu.sync_copy(u4_hbm.at[cg, pl.ds(s * SLAB, SLAB)],
                            acc_sh.at[pl.ds(s * SLAB, SLAB)])

        @pl.when(s == NS - 1)
        def _():
            pltpu.sync_copy(u4_hbm.at[cg, pl.ds(15 * SLAB, SLAB_LAST)],
                            tab_sh.at[pl.ds(15 * SLAB, SLAB_LAST)])
            pltpu.sync_copy(u4_hbm.at[cg, pl.ds(15 * SLAB, SLAB_LAST)],
                            acc_sh.at[pl.ds(15 * SLAB, SLAB_LAST)])

        plsc.subcore_barrier()

        def fire_gathers(h, base):
            for q in range(NBUF):
                pltpu.async_copy(tab_sh.at[src_v.at[base + q]],
                                 rows_v.at[h, q], gsems[h])

        def fire_scatters(h, base):
            for q in range(NBUF):
                pltpu.async_copy(rows_v.at[h, q],
                                 acc_sh.at[dst_v.at[base + q]],
                                 ssems[h], add=True)

        def wait_half(sem):
            pltpu.make_async_copy(u4_hbm.at[:, pl.ds(0, CH)],
                                  rows_v.at[0], sem).wait()

        fire_gathers(0, 0)

        def pair(pi, carry):
            base = pi * 2 * NBUF
            wait_half(gsems[0])
            fire_scatters(0, base)

            @pl.when(pi > 0)
            def _():
                wait_half(ssems[1])

            fire_gathers(1, base + NBUF)
            wait_half(gsems[1])
            fire_scatters(1, base + NBUF)
            wait_half(ssems[0])

            @pl.when(base + 2 * NBUF < K_CH)
            def _():
                fire_gathers(0, base + 2 * NBUF)
            return carry

        lax.fori_loop(0, K_CH // (2 * NBUF), pair, 0)
        wait_half(ssems[1])
        plsc.subcore_barrier()

        @pl.when(s < NS - 1)
        def _():
            pltpu.sync_copy(acc_sh.at[pl.ds(s * SLAB, SLAB)],
                            out_hbm.at[cg, pl.ds(s * SLAB, SLAB)])

        @pl.when(s == NS - 1)
        def _():
            pltpu.sync_copy(acc_sh.at[pl.ds(15 * SLAB, SLAB_LAST)],
                            out_hbm.at[cg, pl.ds(15 * SLAB, SLAB_LAST)])

        plsc.subcore_barrier()


_agg = pl.kernel(
    _agg_body,
    out_type=jax.ShapeDtypeStruct((NCG, N, HG), jnp.float32),
    mesh=plsc.VectorSubcoreMesh(core_axis_name="c", subcore_axis_name="s",
                                num_cores=NC, num_subcores=NS),
    scratch_types=[
        pltpu.VMEM((K_CH, CH), jnp.int32),
        pltpu.VMEM((K_CH, CH), jnp.int32),
        pltpu.VMEM((2, NBUF, CH, HG), jnp.float32),
        pltpu.VMEM_SHARED((N, HG), jnp.float32),
        pltpu.VMEM_SHARED((N_PAD, HG), jnp.float32),
    ] + [pltpu.SemaphoreType.DMA] * 4,
    compiler_params=pltpu.CompilerParams(use_tc_tiling_on_sc=False),
)


DR = 640
R_PER = DR // NS


def _deg_body(dstr_hbm, srcr_hbm, deg_hbm, pmin_hbm, idx_v, tl, buf, red, sh):
    c = lax.axis_index("c")
    s = lax.axis_index("s")

    @pl.when(c == 0)
    def _():
        pltpu.sync_copy(dstr_hbm.at[s], idx_v)
        ones = jnp.ones((16,), jnp.int32)

        def z(i, carry):
            tl[i] = jnp.zeros((16,), jnp.int32)
            return carry

        lax.fori_loop(0, DR, z, 0)

        def body(j, carry):
            for q in range(CH // 16):
                d16 = idx_v[j, pl.ds(q * 16, 16)]
                rows = lax.shift_right_logical(d16, 4)
                cols = lax.bitwise_and(d16, 15)
                plsc.addupdate_scatter(tl, [rows, cols], ones)
            return carry

        lax.fori_loop(0, K_CH, body, 0)
        pltpu.sync_copy(tl, sh.at[s])
        plsc.subcore_barrier()

        def z2(i, carry):
            red[i] = jnp.zeros((16,), jnp.int32)
            return carry

        lax.fori_loop(0, R_PER, z2, 0)

        def rbody(u, carry):
            pltpu.sync_copy(sh.at[u, pl.ds(s * R_PER, R_PER)], buf)

            def add(r, c2):
                red[r] = red[r] + buf[r]
                return c2

            lax.fori_loop(0, R_PER, add, 0)
            return carry

        lax.fori_loop(0, NS, rbody, 0)
        pltpu.sync_copy(red, deg_hbm.at[pl.ds(s * R_PER, R_PER)])

    @pl.when(c == 1)
    def _():
        pltpu.sync_copy(srcr_hbm.at[s], idx_v)
        sent = jnp.full((16,), E_PAD, jnp.int32)

        def z(i, carry):
            tl[i] = sent
            return carry

        lax.fori_loop(0, DR, z, 0)
        iota16 = lax.iota(jnp.int32, 16)

        def body(j, carry):
            for q in range(CH // 16):
                s16 = idx_v[j, pl.ds(q * 16, 16)]
                rows = lax.shift_right_logical(s16, 4)
                cols = lax.bitwise_and(s16, 15)
                pos16 = (s * (K_CH * CH) + j * CH + q * 16) + iota16

                def conv_body(cont):
                    cur = plsc.load_gather(tl, [rows, cols])
                    win = pos16 < cur
                    plsc.store_scatter(tl, [rows, cols], pos16, mask=win)
                    return jnp.any(win)

                lax.while_loop(lambda cont: cont, conv_body, True)
            return carry

        lax.fori_loop(0, K_CH, body, 0)
        pltpu.sync_copy(tl, sh.at[s])
        plsc.subcore_barrier()

        def z2(i, carry):
            red[i] = sent
            return carry

        lax.fori_loop(0, R_PER, z2, 0)

        def rbody(u, carry):
            pltpu.sync_copy(sh.at[u, pl.ds(s * R_PER, R_PER)], buf)

            def mn(r, c2):
                red[r] = jnp.minimum(red[r], buf[r])
                return c2

            lax.fori_loop(0, R_PER, mn, 0)
            return carry

        lax.fori_loop(0, NS, rbody, 0)
        pltpu.sync_copy(red, pmin_hbm.at[pl.ds(s * R_PER, R_PER)])


_deg = pl.kernel(
    _deg_body,
    out_type=(jax.ShapeDtypeStruct((DR, 16), jnp.int32),
              jax.ShapeDtypeStruct((DR, 16), jnp.int32)),
    mesh=plsc.VectorSubcoreMesh(core_axis_name="c", subcore_axis_name="s",
                                num_cores=NC, num_subcores=NS),
    scratch_types=[
        pltpu.VMEM((K_CH, CH), jnp.int32),
        pltpu.VMEM((DR, 16), jnp.int32),
        pltpu.VMEM((R_PER, 16), jnp.int32),
        pltpu.VMEM((R_PER, 16), jnp.int32),
        pltpu.VMEM_SHARED((NS, DR, 16), jnp.int32),
    ],
    compiler_params=pltpu.CompilerParams(use_tc_tiling_on_sc=False,
                                         needs_layout_passes=False),
)


RB = 2000
NB = N // RB


def _bcast_spec(shape):
    return pl.BlockSpec(shape, lambda i: (0,) * len(shape))


def _row_spec(cols):
    return pl.BlockSpec((RB, cols), lambda i: (i, 0))


def _split_spec():
    return pl.BlockSpec((NCG, RB, HG), lambda i: (0, i, 0))


def _split(u):
    return jnp.stack([u[:, g * HG:(g + 1) * HG] for g in range(NCG)], axis=0)


def _unsplit(ref):
    return jnp.concatenate([ref[g] for g in range(NCG)], axis=1)


def _ghost_kernel(xin_ref, w1t, b1, w2t, g1t, gb1, g2t, out_ref):
    xin = xin_ref[...]
    z = jnp.maximum(jnp.dot(xin, w1t[...]) + b1[...], 0.0)
    delta = jnp.dot(z, w2t[...])
    g = jnp.maximum(jnp.dot(xin, g1t[...]) + gb1[...], 0.0)
    gate = jax.nn.sigmoid(jnp.dot(g, g2t[...]))
    out_ref[...] = delta * gate


def _embed_kernel(x_ref, dinv_ref, linw, linb, w0t, out_ref):
    h0 = jnp.maximum(x_ref[...] * linw[...] + linb[...], 0.0)
    out_ref[...] = _split(jnp.dot(dinv_ref[...] * h0, w0t[...]))[None]


def _mid_kernel(s0_ref, dinv_ref, b0, w1t, out_ref):
    sfull = _unsplit(s0_ref)
    h = jnp.maximum(dinv_ref[...] * sfull + b0[...], 0.0)
    out_ref[...] = _split(jnp.dot(dinv_ref[...] * h, w1t[...]))


def _gru_kernel(s1_ref, dinv_ref, b1, wiht, whht, bih, bhh, h_ref, out_ref):
    sfull = _unsplit(s1_ref)
    g = jnp.maximum(dinv_ref[...] * sfull + b1[...], 0.0)
    gi = jnp.dot(g, wiht[...]) + bih[...]
    gh = jnp.dot(h_ref[...], whht[...]) + bhh[...]
    ir, iz, inn = gi[:, :HID], gi[:, HID:2 * HID], gi[:, 2 * HID:]
    hr, hz, hn = gh[:, :HID], gh[:, HID:2 * HID], gh[:, 2 * HID:]
    r = jax.nn.sigmoid(ir + hr)
    z = jax.nn.sigmoid(iz + hz)
    n = jnp.tanh(inn + r * hn)
    out_ref[...] = (1.0 - z) * n + z * h_ref[...]


def _ln(x, g, b):
    m = jnp.mean(x, axis=-1, keepdims=True)
    v = jnp.mean((x - m) ** 2, axis=-1, keepdims=True)
    return (x - m) / jnp.sqrt(v + 1e-05) * g + b


def _pred_kernel(h_ref, w1t, pb1, g1, bb1, w2t, pb2, g2, bb2, p3, pb3,
                 linw, linb, w0t, dinv_ref, y_ref, u0_ref):
    z1 = jnp.maximum(_ln(jnp.dot(h_ref[...], w1t[...]) + pb1[...],
                         g1[...], bb1[...]), 0.0)
    z2 = jnp.maximum(_ln(jnp.dot(z1, w2t[...]) + pb2[...],
                         g2[...], bb2[...]), 0.0)
    y = jnp.sum(z2 * p3[...], axis=1, keepdims=True) + pb3[...]
    y_ref[...] = y
    h0 = jnp.maximum(y * linw[...] + linb[...], 0.0)
    u0_ref[...] = _split(jnp.dot(dinv_ref[...] * h0, w0t[...]))


_ghost_call = pl.pallas_call(
    _ghost_kernel,
    out_shape=jax.ShapeDtypeStruct((T_IN * NG, 128), jnp.float32),
    in_specs=[_bcast_spec((T_IN * NG, 128)), _bcast_spec((128, 128)),
              _bcast_spec((1, 128)), _bcast_spec((128, 128)),
              _bcast_spec((128, 128)), _bcast_spec((1, 128)),
              _bcast_spec((128, 128))],
    out_specs=_bcast_spec((T_IN * NG, 128)),
    grid=(1,),
)

_embed_call = pl.pallas_call(
    _embed_kernel,
    out_shape=jax.ShapeDtypeStruct((T_IN, NCG, N, HG), jnp.float32),
    in_specs=[pl.BlockSpec((RB, 1), lambda t, i: (t * NB + i, 0)),
              pl.BlockSpec((RB, 1), lambda t, i: (i, 0)),
              pl.BlockSpec((1, HID), lambda t, i: (0, 0)),
              pl.BlockSpec((1, HID), lambda t, i: (0, 0)),
              pl.BlockSpec((HID, HID), lambda t, i: (0, 0))],
    out_specs=pl.BlockSpec((1, NCG, RB, HG), lambda t, i: (t, 0, i, 0)),
    grid=(T_IN, NB),
)

_mid_call = pl.pallas_call(
    _mid_kernel,
    out_shape=jax.ShapeDtypeStruct((NCG, N, HG), jnp.float32),
    in_specs=[_split_spec(), _row_spec(1),
              _bcast_spec((1, HID)), _bcast_spec((HID, HID))],
    out_specs=_split_spec(),
    grid=(NB,),
)

_gru_call = pl.pallas_call(
    _gru_kernel,
    out_shape=jax.ShapeDtypeStruct((N, HID), jnp.float32),
    in_specs=[_split_spec(), _row_spec(1),
              _bcast_spec((1, HID)), _bcast_spec((HID, 3 * HID)),
              _bcast_spec((HID, 3 * HID)), _bcast_spec((1, 3 * HID)),
              _bcast_spec((1, 3 * HID)), _row_spec(HID)],
    out_specs=_row_spec(HID),
    grid=(NB,),
)

_pred_call = pl.pallas_call(
    _pred_kernel,
    out_shape=(jax.ShapeDtypeStruct((N, 1), jnp.float32),
               jax.ShapeDtypeStruct((NCG, N, HG), jnp.float32)),
    in_specs=[_row_spec(HID),
              _bcast_spec((HID, HID)), _bcast_spec((1, HID)),
              _bcast_spec((1, HID)), _bcast_spec((1, HID)),
              _bcast_spec((HID, HID)), _bcast_spec((1, HID)),
              _bcast_spec((1, HID)), _bcast_spec((1, HID)),
              _bcast_spec((1, HID)), _bcast_spec((1, 1)),
              _bcast_spec((1, HID)), _bcast_spec((1, HID)),
              _bcast_spec((HID, HID)), _row_spec(1)],
    out_specs=(_row_spec(1), _split_spec()),
    grid=(NB,),
)


def _pad_w(w, shape):
    out = jnp.zeros(shape, jnp.float32)
    return out.at[:w.shape[0], :w.shape[1]].set(w)


def kernel(x, edge_index, edge_attr, mask, params):
    p = params
    src, dst = edge_index[0], edge_index[1]

    pad = E_PAD - E
    dst_r = jnp.concatenate([dst, jnp.full((pad,), N, dst.dtype)]).reshape(NS, K_CH, CH)
    src_r = jnp.concatenate([src, jnp.zeros((pad,), src.dtype)]).reshape(NS, K_CH, CH)
    deg_t, pmin_t = _deg(dst_r, src_r)
    deg = 1.0 + deg_t.reshape(-1)[:N].astype(jnp.float32)
    dinv = 1.0 / jnp.sqrt(deg)
    dinv2 = dinv[:, None]

    pos_min = pmin_t.reshape(-1)[:N]
    has_first = pos_min < E
    first_pos = jnp.minimum(pos_min, E - 1)
    first_out = jnp.where(has_first, dst[first_pos], jnp.array(-1, src.dtype))
    ghost_idx = jnp.arange(0, N, 10)
    dK = []
    cur = first_out[ghost_idx]
    for _ in range(D_DEPTH):
        nxt = first_out[cur]
        nxt = jnp.where(nxt < 0, cur, nxt)
        dK.append(nxt)
        cur = nxt
    dist = jnp.maximum(edge_attr[:, 0], 1e-06)
    first_dx = jnp.where(has_first, dist[first_pos], jnp.float32(-1.0))
    dx_bnd = jnp.maximum(first_dx[ghost_idx], 1e-06)


    X = x[:, :, 0]
    xg = X[0::10]
    d1 = X[dK[0]]
    d2 = X[dK[1]]
    xin = jnp.stack([xg.T, d1.T, d2.T,
                     jnp.broadcast_to(dx_bnd[None, :], (T_IN, NG))], axis=-1)
    xin = xin.reshape(T_IN * NG, 4)
    xin_p = jnp.concatenate([xin, jnp.zeros((T_IN * NG, 124), jnp.float32)], axis=1)
    gf_out = _ghost_call(
        xin_p,
        _pad_w(p['gf_fc1_W'], (128, 128)).T,
        _pad_w(p['gf_fc1_b'][None, :], (1, 128)),
        _pad_w(p['gf_fc2_W'], (128, 128)).T,
        _pad_w(p['gf_g1_W'], (128, 128)).T,
        _pad_w(p['gf_g1_b'][None, :], (1, 128)),
        _pad_w(p['gf_g2_W'], (128, 128)).T,
    )
    dcol = gf_out[:, 0].reshape(T_IN, NG)
    fused = xg.T + ALPHA * dcol
    Xr = X.reshape(NG, 10, T_IN)
    Xf = jnp.concatenate([fused.T[:, None, :], Xr[:, 1:, :]], axis=1).reshape(N, T_IN)

    linw = p['lin_W'][:, 0][None, :]
    linb = p['lin_b'][None, :]
    w0t = p['conv0_W'].T
    w1t = p['conv1_W'].T
    U0_all = _embed_call(Xf.T.reshape(T_IN * N, 1), dinv2, linw, linb, w0t)

    wiht = p['gru_Wih'].T
    whht = p['gru_Whh'].T
    bih = p['gru_bih'][None, :]
    bhh = p['gru_bhh'][None, :]
    b0 = p['conv0_b'][None, :]
    b1 = p['conv1_b'][None, :]

    def step(u0, h):
        s0 = _agg(u0, src_r, dst_r)
        u1 = _mid_call(s0, dinv2, b0, w1t)
        s1 = _agg(u1, src_r, dst_r)
        return _gru_call(s1, dinv2, b1, wiht, whht, bih, bhh, h)

    h = jnp.zeros((N, HID), jnp.float32)
    for t in range(T_IN):
        h = step(U0_all[t], h)

    pred_args = (p['pred_W1'].T, p['pred_b1'][None, :], p['pred_ln1_g'][None, :],
                 p['pred_ln1_b'][None, :], p['pred_W2'].T, p['pred_b2'][None, :],
                 p['pred_ln2_g'][None, :], p['pred_ln2_b'][None, :],
                 p['pred_W3'], p['pred_b3'][None, :], linw, linb, w0t)

    outs = []
    u0 = U0_all[T_IN - 1]
    for k in range(T_OUT):
        h = step(u0, h)
        y, u0 = _pred_call(h, *pred_args, dinv2)
        outs.append(y.reshape(NG, 10)[:, 1:].reshape(-1, 1))
    return jnp.concatenate(outs, axis=1)


# speedup vs baseline: 11.7006x; 1.0073x over previous
import functools

import jax
import jax.numpy as jnp
from jax import lax
from jax.experimental import pallas as pl
from jax.experimental.pallas import tpu as pltpu
from jax.experimental.pallas import tpu_sc as plsc

N = 10000
E = 320000
HID = 128
NCG = 4
HG = HID // NCG
NGC = NCG // 2
T_IN = 8
T_OUT = 4
ALPHA = 0.5
D_DEPTH = 2
NG = N // 10

NC = 2
NS = 16
CH = 128
NBUF = 5
K_CH = -(-E // (NS * CH * 2 * NBUF)) * 2 * NBUF
E_PAD = NS * CH * K_CH
N_PAD = 10048
SLAB = 632
SLAB_LAST = N - 15 * SLAB


def _agg_body(u4_hbm, srcr_hbm, dstr_hbm, out_hbm, src_v, dst_v, rows_v,
              tab_sh, acc_sh, *sems):
    gsems = sems[:2]
    ssems = sems[2:]
    c = lax.axis_index("c")
    s = lax.axis_index("s")
    pltpu.sync_copy(srcr_hbm.at[s], src_v)
    pltpu.sync_copy(dstr_hbm.at[s], dst_v)

    for g in range(NGC):
        cg = c * NGC + g
        @pl.when(s < NS - 1)
        def _():
            pltpu.sync_copy(u4_hbm.at[cg, pl.ds(s * SLAB, SLAB)],
                            tab_sh.at[pl.ds(s * SLAB, SLAB)])
            pltpu.sync_copy(u4_hbm.at[cg, pl.ds(s * SLAB, SLAB)],
                            acc_sh.at[pl.ds(s * SLAB, SLAB)])

        @pl.when(s == NS - 1)
        def _():
            pltpu.sync_copy(u4_hbm.at[cg, pl.ds(15 * SLAB, SLAB_LAST)],
                            tab_sh.at[pl.ds(15 * SLAB, SLAB_LAST)])
            pltpu.sync_copy(u4_hbm.at[cg, pl.ds(15 * SLAB, SLAB_LAST)],
                            acc_sh.at[pl.ds(15 * SLAB, SLAB_LAST)])

        plsc.subcore_barrier()

        def fire_gathers(h, base):
            for q in range(NBUF):
                pltpu.async_copy(tab_sh.at[src_v.at[base + q]],
                                 rows_v.at[h, q], gsems[h])

        def fire_scatters(h, base):
            for q in range(NBUF):
                pltpu.async_copy(rows_v.at[h, q],
                                 acc_sh.at[dst_v.at[base + q]],
                                 ssems[h], add=True)

        def wait_half(sem):
            pltpu.make_async_copy(u4_hbm.at[:, pl.ds(0, CH)],
                                  rows_v.at[0], sem).wait()

        fire_gathers(0, 0)

        def pair(pi, carry):
            base = pi * 2 * NBUF
            wait_half(gsems[0])
            fire_scatters(0, base)

            @pl.when(pi > 0)
            def _():
                wait_half(ssems[1])

            fire_gathers(1, base + NBUF)
            wait_half(gsems[1])
            fire_scatters(1, base + NBUF)
            wait_half(ssems[0])

            @pl.when(base + 2 * NBUF < K_CH)
            def _():
                fire_gathers(0, base + 2 * NBUF)
            return carry

        lax.fori_loop(0, K_CH // (2 * NBUF), pair, 0)
        wait_half(ssems[1])
        plsc.subcore_barrier()

        @pl.when(s < NS - 1)
        def _():
            pltpu.sync_copy(acc_sh.at[pl.ds(s * SLAB, SLAB)],
                            out_hbm.at[cg, pl.ds(s * SLAB, SLAB)])

        @pl.when(s == NS - 1)
        def _():
            pltpu.sync_copy(acc_sh.at[pl.ds(15 * SLAB, SLAB_LAST)],
                            out_hbm.at[cg, pl.ds(15 * SLAB, SLAB_LAST)])

        plsc.subcore_barrier()


_agg = pl.kernel(
    _agg_body,
    out_type=jax.ShapeDtypeStruct((NCG, N, HG), jnp.float32),
    mesh=plsc.VectorSubcoreMesh(core_axis_name="c", subcore_axis_name="s",
                                num_cores=NC, num_subcores=NS),
    scratch_types=[
        pltpu.VMEM((K_CH, CH), jnp.int32),
        pltpu.VMEM((K_CH, CH), jnp.int32),
        pltpu.VMEM((2, NBUF, CH, HG), jnp.float32),
        pltpu.VMEM_SHARED((N, HG), jnp.float32),
        pltpu.VMEM_SHARED((N_PAD, HG), jnp.float32),
    ] + [pltpu.SemaphoreType.DMA] * 4,
    compiler_params=pltpu.CompilerParams(use_tc_tiling_on_sc=False),
)


DR = 640
R_PER = DR // NS


def _deg_body(dstr_hbm, srcr_hbm, deg_hbm, pmin_hbm, idx_v, tl, buf, red, sh):
    c = lax.axis_index("c")
    s = lax.axis_index("s")

    @pl.when(c == 0)
    def _():
        pltpu.sync_copy(dstr_hbm.at[s], idx_v)
        ones = jnp.ones((16,), jnp.int32)

        def z(i, carry):
            tl[i] = jnp.zeros((16,), jnp.int32)
            return carry

        lax.fori_loop(0, DR, z, 0)

        def body(j, carry):
            for q in range(CH // 16):
                d16 = idx_v[j, pl.ds(q * 16, 16)]
                rows = lax.shift_right_logical(d16, 4)
                cols = lax.bitwise_and(d16, 15)
                plsc.addupdate_scatter(tl, [rows, cols], ones)
            return carry

        lax.fori_loop(0, K_CH, body, 0)
        pltpu.sync_copy(tl, sh.at[s])
        plsc.subcore_barrier()

        def z2(i, carry):
            red[i] = jnp.zeros((16,), jnp.int32)
            return carry

        lax.fori_loop(0, R_PER, z2, 0)

        def rbody(u, carry):
            pltpu.sync_copy(sh.at[u, pl.ds(s * R_PER, R_PER)], buf)

            def add(r, c2):
                red[r] = red[r] + buf[r]
                return c2

            lax.fori_loop(0, R_PER, add, 0)
            return carry

        lax.fori_loop(0, NS, rbody, 0)
        pltpu.sync_copy(red, deg_hbm.at[pl.ds(s * R_PER, R_PER)])

    @pl.when(c == 1)
    def _():
        pltpu.sync_copy(srcr_hbm.at[s], idx_v)
        sent = jnp.full((16,), E_PAD, jnp.int32)

        def z(i, carry):
            tl[i] = sent
            return carry

        lax.fori_loop(0, DR, z, 0)
        iota16 = lax.iota(jnp.int32, 16)

        def body(j, carry):
            for q in range(CH // 16):
                s16 = idx_v[j, pl.ds(q * 16, 16)]
                rows = lax.shift_right_logical(s16, 4)
                cols = lax.bitwise_and(s16, 15)
                pos16 = (s * (K_CH * CH) + j * CH + q * 16) + iota16

                def conv_body(cont):
                    cur = plsc.load_gather(tl, [rows, cols])
                    win = pos16 < cur
                    plsc.store_scatter(tl, [rows, cols], pos16, mask=win)
                    return jnp.any(win)

                lax.while_loop(lambda cont: cont, conv_body, True)
            return carry

        lax.fori_loop(0, K_CH, body, 0)
        pltpu.sync_copy(tl, sh.at[s])
        plsc.subcore_barrier()

        def z2(i, carry):
            red[i] = sent
            return carry

        lax.fori_loop(0, R_PER, z2, 0)

        def rbody(u, carry):
            pltpu.sync_copy(sh.at[u, pl.ds(s * R_PER, R_PER)], buf)

            def mn(r, c2):
                red[r] = jnp.minimum(red[r], buf[r])
                return c2

            lax.fori_loop(0, R_PER, mn, 0)
            return carry

        lax.fori_loop(0, NS, rbody, 0)
        pltpu.sync_copy(red, pmin_hbm.at[pl.ds(s * R_PER, R_PER)])


_deg = pl.kernel(
    _deg_body,
    out_type=(jax.ShapeDtypeStruct((DR, 16), jnp.int32),
              jax.ShapeDtypeStruct((DR, 16), jnp.int32)),
    mesh=plsc.VectorSubcoreMesh(core_axis_name="c", subcore_axis_name="s",
                                num_cores=NC, num_subcores=NS),
    scratch_types=[
        pltpu.VMEM((K_CH, CH), jnp.int32),
        pltpu.VMEM((DR, 16), jnp.int32),
        pltpu.VMEM((R_PER, 16), jnp.int32),
        pltpu.VMEM((R_PER, 16), jnp.int32),
        pltpu.VMEM_SHARED((NS, DR, 16), jnp.int32),
    ],
    compiler_params=pltpu.CompilerParams(use_tc_tiling_on_sc=False,
                                         needs_layout_passes=False),
)


RB = 2000
NB = N // RB


def _bcast_spec(shape):
    return pl.BlockSpec(shape, lambda i: (0,) * len(shape))


def _row_spec(cols):
    return pl.BlockSpec((RB, cols), lambda i: (i, 0))


def _split_spec():
    return pl.BlockSpec((NCG, RB, HG), lambda i: (0, i, 0))


def _split(u):
    return jnp.stack([u[:, g * HG:(g + 1) * HG] for g in range(NCG)], axis=0)


def _unsplit(ref):
    return jnp.concatenate([ref[g] for g in range(NCG)], axis=1)


def _ghost_kernel(xin_ref, w1t, b1, w2t, g1t, gb1, g2t, out_ref):
    xin = xin_ref[...]
    z = jnp.maximum(jnp.dot(xin, w1t[...]) + b1[...], 0.0)
    delta = jnp.dot(z, w2t[...])
    g = jnp.maximum(jnp.dot(xin, g1t[...]) + gb1[...], 0.0)
    gate = jax.nn.sigmoid(jnp.dot(g, g2t[...]))
    out_ref[...] = delta * gate


def _embed_kernel(x_ref, dinv_ref, linw, linb, w0t, out_ref):
    h0 = jnp.maximum(x_ref[...] * linw[...] + linb[...], 0.0)
    out_ref[...] = _split(jnp.dot(dinv_ref[...] * h0, w0t[...]))[None]


def _mid_kernel(s0_ref, dinv_ref, b0, w1t, out_ref):
    sfull = _unsplit(s0_ref)
    h = jnp.maximum(dinv_ref[...] * sfull + b0[...], 0.0)
    out_ref[...] = _split(jnp.dot(dinv_ref[...] * h, w1t[...]))


def _gru_kernel(s1_ref, dinv_ref, b1, wiht, whht, bih, bhh, h_ref, out_ref):
    sfull = _unsplit(s1_ref)
    g = jnp.maximum(dinv_ref[...] * sfull + b1[...], 0.0)
    gi = jnp.dot(g, wiht[...]) + bih[...]
    gh = jnp.dot(h_ref[...], whht[...]) + bhh[...]
    ir, iz, inn = gi[:, :HID], gi[:, HID:2 * HID], gi[:, 2 * HID:]
    hr, hz, hn = gh[:, :HID], gh[:, HID:2 * HID], gh[:, 2 * HID:]
    r = jax.nn.sigmoid(ir + hr)
    z = jax.nn.sigmoid(iz + hz)
    n = jnp.tanh(inn + r * hn)
    out_ref[...] = (1.0 - z) * n + z * h_ref[...]


def _ln(x, g, b):
    m = jnp.mean(x, axis=-1, keepdims=True)
    v = jnp.mean((x - m) ** 2, axis=-1, keepdims=True)
    return (x - m) / jnp.sqrt(v + 1e-05) * g + b


def _pred_kernel(h_ref, w1t, pb1, g1, bb1, w2t, pb2, g2, bb2, p3, pb3,
                 linw, linb, w0t, dinv_ref, y_ref, u0_ref):
    z1 = jnp.maximum(_ln(jnp.dot(h_ref[...], w1t[...]) + pb1[...],
                         g1[...], bb1[...]), 0.0)
    z2 = jnp.maximum(_ln(jnp.dot(z1, w2t[...]) + pb2[...],
                         g2[...], bb2[...]), 0.0)
    y = jnp.sum(z2 * p3[...], axis=1, keepdims=True) + pb3[...]
    y_ref[...] = y
    h0 = jnp.maximum(y * linw[...] + linb[...], 0.0)
    u0_ref[...] = _split(jnp.dot(dinv_ref[...] * h0, w0t[...]))


_ghost_call = pl.pallas_call(
    _ghost_kernel,
    out_shape=jax.ShapeDtypeStruct((T_IN * NG, 128), jnp.float32),
    in_specs=[_bcast_spec((T_IN * NG, 128)), _bcast_spec((128, 128)),
              _bcast_spec((1, 128)), _bcast_spec((128, 128)),
              _bcast_spec((128, 128)), _bcast_spec((1, 128)),
              _bcast_spec((128, 128))],
    out_specs=_bcast_spec((T_IN * NG, 128)),
    grid=(1,),
)

_embed_call = pl.pallas_call(
    _embed_kernel,
    out_shape=jax.ShapeDtypeStruct((T_IN, NCG, N, HG), jnp.float32),
    in_specs=[pl.BlockSpec((RB, 1), lambda t, i: (t * NB + i, 0)),
              pl.BlockSpec((RB, 1), lambda t, i: (i, 0)),
              pl.BlockSpec((1, HID), lambda t, i: (0, 0)),
              pl.BlockSpec((1, HID), lambda t, i: (0, 0)),
              pl.BlockSpec((HID, HID), lambda t, i: (0, 0))],
    out_specs=pl.BlockSpec((1, NCG, RB, HG), lambda t, i: (t, 0, i, 0)),
    grid=(T_IN, NB),
)

_mid_call = pl.pallas_call(
    _mid_kernel,
    out_shape=jax.ShapeDtypeStruct((NCG, N, HG), jnp.float32),
    in_specs=[_split_spec(), _row_spec(1),
              _bcast_spec((1, HID)), _bcast_spec((HID, HID))],
    out_specs=_split_spec(),
    grid=(NB,),
)

_gru_call = pl.pallas_call(
    _gru_kernel,
    out_shape=jax.ShapeDtypeStruct((N, HID), jnp.float32),
    in_specs=[_split_spec(), _row_spec(1),
              _bcast_spec((1, HID)), _bcast_spec((HID, 3 * HID)),
              _bcast_spec((HID, 3 * HID)), _bcast_spec((1, 3 * HID)),
              _bcast_spec((1, 3 * HID)), _row_spec(HID)],
    out_specs=_row_spec(HID),
    grid=(NB,),
)

_pred_call = pl.pallas_call(
    _pred_kernel,
    out_shape=(jax.ShapeDtypeStruct((N, 1), jnp.float32),
               jax.ShapeDtypeStruct((NCG, N, HG), jnp.float32)),
    in_specs=[_row_spec(HID),
              _bcast_spec((HID, HID)), _bcast_spec((1, HID)),
              _bcast_spec((1, HID)), _bcast_spec((1, HID)),
              _bcast_spec((HID, HID)), _bcast_spec((1, HID)),
              _bcast_spec((1, HID)), _bcast_spec((1, HID)),
              _bcast_spec((1, HID)), _bcast_spec((1, 1)),
              _bcast_spec((1, HID)), _bcast_spec((1, HID)),
              _bcast_spec((HID, HID)), _row_spec(1)],
    out_specs=(_row_spec(1), _split_spec()),
    grid=(NB,),
)


def _pad_w(w, shape):
    out = jnp.zeros(shape, jnp.float32)
    return out.at[:w.shape[0], :w.shape[1]].set(w)


def kernel(x, edge_index, edge_attr, mask, params):
    p = params
    src, dst = edge_index[0], edge_index[1]

    pad = E_PAD - E
    dst_r = jnp.concatenate([dst, jnp.full((pad,), N, dst.dtype)]).reshape(NS, K_CH, CH)
    src_r = jnp.concatenate([src, jnp.zeros((pad,), src.dtype)]).reshape(NS, K_CH, CH)
    deg_t, pmin_t = _deg(dst_r, src_r)
    deg = 1.0 + deg_t.reshape(-1)[:N].astype(jnp.float32)
    dinv = 1.0 / jnp.sqrt(deg)
    dinv2 = dinv[:, None]

    pos_min = pmin_t.reshape(-1)[:N]
    has_first = pos_min < E
    first_pos = jnp.minimum(pos_min, E - 1)
    first_out = jnp.where(has_first, dst[first_pos], jnp.array(-1, src.dtype))
    ghost_idx = jnp.arange(0, N, 10)
    dK = []
    cur = first_out[ghost_idx]
    for _ in range(D_DEPTH):
        nxt = first_out[cur]
        nxt = jnp.where(nxt < 0, cur, nxt)
        dK.append(nxt)
        cur = nxt
    dist = jnp.maximum(edge_attr[:, 0], 1e-06)
    first_dx = jnp.where(has_first, dist[first_pos], jnp.float32(-1.0))
    dx_bnd = jnp.maximum(first_dx[ghost_idx], 1e-06)


    X = x[:, :, 0]
    xg = X[0::10]
    d1 = X[dK[0]]
    d2 = X[dK[1]]
    xin = jnp.stack([xg.T, d1.T, d2.T,
                     jnp.broadcast_to(dx_bnd[None, :], (T_IN, NG))], axis=-1)
    xin = xin.reshape(T_IN * NG, 4)
    xin_p = jnp.concatenate([xin, jnp.zeros((T_IN * NG, 124), jnp.float32)], axis=1)
    gf_out = _ghost_call(
        xin_p,
        _pad_w(p['gf_fc1_W'], (128, 128)).T,
        _pad_w(p['gf_fc1_b'][None, :], (1, 128)),
        _pad_w(p['gf_fc2_W'], (128, 128)).T,
        _pad_w(p['gf_g1_W'], (128, 128)).T,
        _pad_w(p['gf_g1_b'][None, :], (1, 128)),
        _pad_w(p['gf_g2_W'], (128, 128)).T,
    )
    dcol = gf_out[:, 0].reshape(T_IN, NG)
    fused = xg.T + ALPHA * dcol
    Xr = X.reshape(NG, 10, T_IN)
    Xf = jnp.concatenate([fused.T[:, None, :], Xr[:, 1:, :]], axis=1).reshape(N, T_IN)

    linw = p['lin_W'][:, 0][None, :]
    linb = p['lin_b'][None, :]
    w0t = p['conv0_W'].T
    w1t = p['conv1_W'].T
    U0_all = _embed_call(Xf.T.reshape(T_IN * N, 1), dinv2, linw, linb, w0t)

    wiht = p['gru_Wih'].T
    whht = p['gru_Whh'].T
    bih = p['gru_bih'][None, :]
    bhh = p['gru_bhh'][None, :]
    b0 = p['conv0_b'][None, :]
    b1 = p['conv1_b'][None, :]

    def step(u0, h):
        s0 = _agg(u0, src_r, dst_r)
        u1 = _mid_call(s0, dinv2, b0, w1t)
        s1 = _agg(u1, src_r, dst_r)
        return _gru_call(s1, dinv2, b1, wiht, whht, bih, bhh, h)

    h = jnp.zeros((N, HID), jnp.float32)
    for t in range(T_IN):
        h = step(U0_all[t], h)

    pred_args = (p['pred_W1'].T, p['pred_b1'][None, :], p['pred_ln1_g'][None, :],
                 p['pred_ln1_b'][None, :], p['pred_W2'].T, p['pred_b2'][None, :],
                 p['pred_ln2_g'][None, :], p['pred_ln2_b'][None, :],
                 p['pred_W3'], p['pred_b3'][None, :], linw, linb, w0t)

    outs = []
    u0 = U0_all[T_IN - 1]
    for k in range(T_OUT):
        h = step(u0, h)
        y, u0 = _pred_call(h, *pred_args, dinv2)
        outs.append(y.reshape(NG, 10)[:, 1:].reshape(-1, 1))
    return jnp.concatenate(outs, axis=1)


# speedup vs baseline: 11.7080x; 1.0006x over previous
import jax
import jax.numpy as jnp
from jax import lax
from jax.experimental import pallas as pl
from jax.experimental.pallas import tpu as pltpu
from jax.experimental.pallas import tpu_sc as plsc

N = 10000
E = 320000
HID = 128
NCG = 4
HG = HID // NCG
NGC = NCG // 2
T_IN = 8
T_OUT = 4
ALPHA = 0.5
D_DEPTH = 2
NG = N // 10

NC = 2
NS = 16
CH = 128
NBUF = 5
K_CH = -(-E // (NS * CH * 2 * NBUF)) * 2 * NBUF
E_PAD = NS * CH * K_CH
N_PAD = 10048
SLAB = 632
SLAB_LAST = N - 15 * SLAB


def _agg_body(u4_hbm, srcr_hbm, dstr_hbm, out_hbm, src_v, dst_v, rows_v,
              tab_sh, acc_sh, *sems):
    gsems = sems[:2]
    ssems = sems[2:]
    c = lax.axis_index("c")
    s = lax.axis_index("s")
    pltpu.sync_copy(srcr_hbm.at[s], src_v)
    pltpu.sync_copy(dstr_hbm.at[s], dst_v)

    for g in range(NGC):
        cg = c * NGC + g
        @pl.when(s < NS - 1)
        def _():
            pltpu.sync_copy(u4_hbm.at[cg, pl.ds(s * SLAB, SLAB)],
                            tab_sh.at[pl.ds(s * SLAB, SLAB)])
            pltpu.sync_copy(u4_hbm.at[cg, pl.ds(s * SLAB, SLAB)],
                            acc_sh.at[pl.ds(s * SLAB, SLAB)])

        @pl.when(s == NS - 1)
        def _():
            pltpu.sync_copy(u4_hbm.at[cg, pl.ds(15 * SLAB, SLAB_LAST)],
                            tab_sh.at[pl.ds(15 * SLAB, SLAB_LAST)])
            pltpu.sync_copy(u4_hbm.at[cg, pl.ds(15 * SLAB, SLAB_LAST)],
                            acc_sh.at[pl.ds(15 * SLAB, SLAB_LAST)])

        plsc.subcore_barrier()

        def fire_gathers(h, base):
            for q in range(NBUF):
                pltpu.async_copy(tab_sh.at[src_v.at[base + q]],
                                 rows_v.at[h, q], gsems[h])

        def fire_scatters(h, base):
            for q in range(NBUF):
                pltpu.async_copy(rows_v.at[h, q],
                                 acc_sh.at[dst_v.at[base + q]],
                                 ssems[h], add=True)

        def wait_half(sem):
            pltpu.make_async_copy(u4_hbm.at[:, pl.ds(0, CH)],
                                  rows_v.at[0], sem).wait()

        fire_gathers(0, 0)

        def pair(pi, carry):
            base = pi * 2 * NBUF
            wait_half(gsems[0])
            fire_scatters(0, base)

            @pl.when(pi > 0)
            def _():
                wait_half(ssems[1])

            fire_gathers(1, base + NBUF)
            wait_half(gsems[1])
            fire_scatters(1, base + NBUF)
            wait_half(ssems[0])

            @pl.when(base + 2 * NBUF < K_CH)
            def _():
                fire_gathers(0, base + 2 * NBUF)
            return carry

        lax.fori_loop(0, K_CH // (2 * NBUF), pair, 0)
        wait_half(ssems[1])
        plsc.subcore_barrier()

        @pl.when(s < NS - 1)
        def _():
            pltpu.sync_copy(acc_sh.at[pl.ds(s * SLAB, SLAB)],
                            out_hbm.at[cg, pl.ds(s * SLAB, SLAB)])

        @pl.when(s == NS - 1)
        def _():
            pltpu.sync_copy(acc_sh.at[pl.ds(15 * SLAB, SLAB_LAST)],
                            out_hbm.at[cg, pl.ds(15 * SLAB, SLAB_LAST)])

        plsc.subcore_barrier()


_agg = pl.kernel(
    _agg_body,
    out_type=jax.ShapeDtypeStruct((NCG, N, HG), jnp.float32),
    mesh=plsc.VectorSubcoreMesh(core_axis_name="c", subcore_axis_name="s",
                                num_cores=NC, num_subcores=NS),
    scratch_types=[
        pltpu.VMEM((K_CH, CH), jnp.int32),
        pltpu.VMEM((K_CH, CH), jnp.int32),
        pltpu.VMEM((2, NBUF, CH, HG), jnp.float32),
        pltpu.VMEM_SHARED((N, HG), jnp.float32),
        pltpu.VMEM_SHARED((N_PAD, HG), jnp.float32),
    ] + [pltpu.SemaphoreType.DMA] * 4,
    compiler_params=pltpu.CompilerParams(use_tc_tiling_on_sc=False),
)


DR = 640
R_PER = DR // NS


def _deg_body(dstr_hbm, srcr_hbm, deg_hbm, pmin_hbm, idx_v, tl, buf, red, sh):
    c = lax.axis_index("c")
    s = lax.axis_index("s")

    @pl.when(c == 0)
    def _():
        pltpu.sync_copy(dstr_hbm.at[s], idx_v)
        ones = jnp.ones((16,), jnp.int32)

        def z(i, carry):
            tl[i] = jnp.zeros((16,), jnp.int32)
            return carry

        lax.fori_loop(0, DR, z, 0)

        def body(j, carry):
            for q in range(CH // 16):
                d16 = idx_v[j, pl.ds(q * 16, 16)]
                rows = lax.shift_right_logical(d16, 4)
                cols = lax.bitwise_and(d16, 15)
                plsc.addupdate_scatter(tl, [rows, cols], ones)
            return carry

        lax.fori_loop(0, K_CH, body, 0)
        pltpu.sync_copy(tl, sh.at[s])
        plsc.subcore_barrier()

        def z2(i, carry):
            red[i] = jnp.zeros((16,), jnp.int32)
            return carry

        lax.fori_loop(0, R_PER, z2, 0)

        def rbody(u, carry):
            pltpu.sync_copy(sh.at[u, pl.ds(s * R_PER, R_PER)], buf)

            def add(r, c2):
                red[r] = red[r] + buf[r]
                return c2

            lax.fori_loop(0, R_PER, add, 0)
            return carry

        lax.fori_loop(0, NS, rbody, 0)
        pltpu.sync_copy(red, deg_hbm.at[pl.ds(s * R_PER, R_PER)])

    @pl.when(c == 1)
    def _():
        pltpu.sync_copy(srcr_hbm.at[s], idx_v)
        sent = jnp.full((16,), E_PAD, jnp.int32)

        def z(i, carry):
            tl[i] = sent
            return carry

        lax.fori_loop(0, DR, z, 0)
        iota16 = lax.iota(jnp.int32, 16)

        def body(j, carry):
            for q in range(CH // 16):
                s16 = idx_v[j, pl.ds(q * 16, 16)]
                rows = lax.shift_right_logical(s16, 4)
                cols = lax.bitwise_and(s16, 15)
                pos16 = (s * (K_CH * CH) + j * CH + q * 16) + iota16

                def conv_body(cont):
                    cur = plsc.load_gather(tl, [rows, cols])
                    win = pos16 < cur
                    plsc.store_scatter(tl, [rows, cols], pos16, mask=win)
                    return jnp.any(win)

                lax.while_loop(lambda cont: cont, conv_body, True)
            return carry

        lax.fori_loop(0, K_CH, body, 0)
        pltpu.sync_copy(tl, sh.at[s])
        plsc.subcore_barrier()

        def z2(i, carry):
            red[i] = sent
            return carry

        lax.fori_loop(0, R_PER, z2, 0)

        def rbody(u, carry):
            pltpu.sync_copy(sh.at[u, pl.ds(s * R_PER, R_PER)], buf)

            def mn(r, c2):
                red[r] = jnp.minimum(red[r], buf[r])
                return c2

            lax.fori_loop(0, R_PER, mn, 0)
            return carry

        lax.fori_loop(0, NS, rbody, 0)
        pltpu.sync_copy(red, pmin_hbm.at[pl.ds(s * R_PER, R_PER)])


_deg = pl.kernel(
    _deg_body,
    out_type=(jax.ShapeDtypeStruct((DR, 16), jnp.int32),
              jax.ShapeDtypeStruct((DR, 16), jnp.int32)),
    mesh=plsc.VectorSubcoreMesh(core_axis_name="c", subcore_axis_name="s",
                                num_cores=NC, num_subcores=NS),
    scratch_types=[
        pltpu.VMEM((K_CH, CH), jnp.int32),
        pltpu.VMEM((DR, 16), jnp.int32),
        pltpu.VMEM((R_PER, 16), jnp.int32),
        pltpu.VMEM((R_PER, 16), jnp.int32),
        pltpu.VMEM_SHARED((NS, DR, 16), jnp.int32),
    ],
    compiler_params=pltpu.CompilerParams(use_tc_tiling_on_sc=False,
                                         needs_layout_passes=False),
)


RB = 2000
NB = N // RB


def _bcast_spec(shape):
    return pl.BlockSpec(shape, lambda i: (0,) * len(shape))


def _row_spec(cols):
    return pl.BlockSpec((RB, cols), lambda i: (i, 0))


def _split_spec():
    return pl.BlockSpec((NCG, RB, HG), lambda i: (0, i, 0))


def _split(u):
    return jnp.stack([u[:, g * HG:(g + 1) * HG] for g in range(NCG)], axis=0)


def _unsplit(ref):
    return jnp.concatenate([ref[g] for g in range(NCG)], axis=1)


def _ghost_kernel(xin_ref, w1t, b1, w2t, g1t, gb1, g2t, out_ref):
    xin = xin_ref[...]
    z = jnp.maximum(jnp.dot(xin, w1t[...]) + b1[...], 0.0)
    delta = jnp.dot(z, w2t[...])
    g = jnp.maximum(jnp.dot(xin, g1t[...]) + gb1[...], 0.0)
    gate = jax.nn.sigmoid(jnp.dot(g, g2t[...]))
    out_ref[...] = delta * gate


def _embed_kernel(x_ref, dinv_ref, linw, linb, w0t, out_ref):
    h0 = jnp.maximum(x_ref[...] * linw[...] + linb[...], 0.0)
    out_ref[...] = _split(jnp.dot(dinv_ref[...] * h0, w0t[...]))[None]


def _mid_kernel(s0_ref, dinv_ref, b0, w1t, out_ref):
    sfull = _unsplit(s0_ref)
    h = jnp.maximum(dinv_ref[...] * sfull + b0[...], 0.0)
    out_ref[...] = _split(jnp.dot(dinv_ref[...] * h, w1t[...]))


def _gru_kernel(s1_ref, dinv_ref, b1, wiht, whht, bih, bhh, h_ref, out_ref):
    sfull = _unsplit(s1_ref)
    g = jnp.maximum(dinv_ref[...] * sfull + b1[...], 0.0)
    gi = jnp.dot(g, wiht[...]) + bih[...]
    gh = jnp.dot(h_ref[...], whht[...]) + bhh[...]
    ir, iz, inn = gi[:, :HID], gi[:, HID:2 * HID], gi[:, 2 * HID:]
    hr, hz, hn = gh[:, :HID], gh[:, HID:2 * HID], gh[:, 2 * HID:]
    r = jax.nn.sigmoid(ir + hr)
    z = jax.nn.sigmoid(iz + hz)
    n = jnp.tanh(inn + r * hn)
    out_ref[...] = (1.0 - z) * n + z * h_ref[...]


def _ln(x, g, b):
    m = jnp.mean(x, axis=-1, keepdims=True)
    v = jnp.mean((x - m) ** 2, axis=-1, keepdims=True)
    return (x - m) / jnp.sqrt(v + 1e-05) * g + b


def _pred_kernel(h_ref, w1t, pb1, g1, bb1, w2t, pb2, g2, bb2, p3, pb3,
                 linw, linb, w0t, dinv_ref, y_ref, u0_ref):
    z1 = jnp.maximum(_ln(jnp.dot(h_ref[...], w1t[...]) + pb1[...],
                         g1[...], bb1[...]), 0.0)
    z2 = jnp.maximum(_ln(jnp.dot(z1, w2t[...]) + pb2[...],
                         g2[...], bb2[...]), 0.0)
    y = jnp.sum(z2 * p3[...], axis=1, keepdims=True) + pb3[...]
    y_ref[...] = y
    h0 = jnp.maximum(y * linw[...] + linb[...], 0.0)
    u0_ref[...] = _split(jnp.dot(dinv_ref[...] * h0, w0t[...]))


_ghost_call = pl.pallas_call(
    _ghost_kernel,
    out_shape=jax.ShapeDtypeStruct((T_IN * NG, 128), jnp.float32),
    in_specs=[_bcast_spec((T_IN * NG, 128)), _bcast_spec((128, 128)),
              _bcast_spec((1, 128)), _bcast_spec((128, 128)),
              _bcast_spec((128, 128)), _bcast_spec((1, 128)),
              _bcast_spec((128, 128))],
    out_specs=_bcast_spec((T_IN * NG, 128)),
    grid=(1,),
)

_embed_call = pl.pallas_call(
    _embed_kernel,
    out_shape=jax.ShapeDtypeStruct((T_IN, NCG, N, HG), jnp.float32),
    in_specs=[pl.BlockSpec((RB, 1), lambda t, i: (t * NB + i, 0)),
              pl.BlockSpec((RB, 1), lambda t, i: (i, 0)),
              pl.BlockSpec((1, HID), lambda t, i: (0, 0)),
              pl.BlockSpec((1, HID), lambda t, i: (0, 0)),
              pl.BlockSpec((HID, HID), lambda t, i: (0, 0))],
    out_specs=pl.BlockSpec((1, NCG, RB, HG), lambda t, i: (t, 0, i, 0)),
    grid=(T_IN, NB),
)

_mid_call = pl.pallas_call(
    _mid_kernel,
    out_shape=jax.ShapeDtypeStruct((NCG, N, HG), jnp.float32),
    in_specs=[_split_spec(), _row_spec(1),
              _bcast_spec((1, HID)), _bcast_spec((HID, HID))],
    out_specs=_split_spec(),
    grid=(NB,),
)

_gru_call = pl.pallas_call(
    _gru_kernel,
    out_shape=jax.ShapeDtypeStruct((N, HID), jnp.float32),
    in_specs=[_split_spec(), _row_spec(1),
              _bcast_spec((1, HID)), _bcast_spec((HID, 3 * HID)),
              _bcast_spec((HID, 3 * HID)), _bcast_spec((1, 3 * HID)),
              _bcast_spec((1, 3 * HID)), _row_spec(HID)],
    out_specs=_row_spec(HID),
    grid=(NB,),
)

_pred_call = pl.pallas_call(
    _pred_kernel,
    out_shape=(jax.ShapeDtypeStruct((N, 1), jnp.float32),
               jax.ShapeDtypeStruct((NCG, N, HG), jnp.float32)),
    in_specs=[_row_spec(HID),
              _bcast_spec((HID, HID)), _bcast_spec((1, HID)),
              _bcast_spec((1, HID)), _bcast_spec((1, HID)),
              _bcast_spec((HID, HID)), _bcast_spec((1, HID)),
              _bcast_spec((1, HID)), _bcast_spec((1, HID)),
              _bcast_spec((1, HID)), _bcast_spec((1, 1)),
              _bcast_spec((1, HID)), _bcast_spec((1, HID)),
              _bcast_spec((HID, HID)), _row_spec(1)],
    out_specs=(_row_spec(1), _split_spec()),
    grid=(NB,),
)


def _pad_w(w, shape):
    out = jnp.zeros(shape, jnp.float32)
    return out.at[:w.shape[0], :w.shape[1]].set(w)


def kernel(x, edge_index, edge_attr, mask, params):
    p = params
    src, dst = edge_index[0], edge_index[1]

    pad = E_PAD - E
    dst_r = jnp.concatenate([dst, jnp.full((pad,), N, dst.dtype)]).reshape(NS, K_CH, CH)
    src_r = jnp.concatenate([src, jnp.zeros((pad,), src.dtype)]).reshape(NS, K_CH, CH)
    deg_t, pmin_t = _deg(dst_r, src_r)
    deg = 1.0 + deg_t.reshape(-1)[:N].astype(jnp.float32)
    dinv = 1.0 / jnp.sqrt(deg)
    dinv2 = dinv[:, None]

    pos_min = pmin_t.reshape(-1)[:N]
    has_first = pos_min < E
    first_pos = jnp.minimum(pos_min, E - 1)
    first_out = jnp.where(has_first, dst[first_pos], jnp.array(-1, src.dtype))
    ghost_idx = jnp.arange(0, N, 10)
    dK = []
    cur = first_out[ghost_idx]
    for _ in range(D_DEPTH):
        nxt = first_out[cur]
        nxt = jnp.where(nxt < 0, cur, nxt)
        dK.append(nxt)
        cur = nxt
    dist = jnp.maximum(edge_attr[:, 0], 1e-06)
    first_dx = jnp.where(has_first, dist[first_pos], jnp.float32(-1.0))
    dx_bnd = jnp.maximum(first_dx[ghost_idx], 1e-06)


    X = x[:, :, 0]
    xg = X[0::10]
    d1 = X[dK[0]]
    d2 = X[dK[1]]
    xin = jnp.stack([xg.T, d1.T, d2.T,
                     jnp.broadcast_to(dx_bnd[None, :], (T_IN, NG))], axis=-1)
    xin = xin.reshape(T_IN * NG, 4)
    xin_p = jnp.concatenate([xin, jnp.zeros((T_IN * NG, 124), jnp.float32)], axis=1)
    gf_out = _ghost_call(
        xin_p,
        _pad_w(p['gf_fc1_W'], (128, 128)).T,
        _pad_w(p['gf_fc1_b'][None, :], (1, 128)),
        _pad_w(p['gf_fc2_W'], (128, 128)).T,
        _pad_w(p['gf_g1_W'], (128, 128)).T,
        _pad_w(p['gf_g1_b'][None, :], (1, 128)),
        _pad_w(p['gf_g2_W'], (128, 128)).T,
    )
    dcol = gf_out[:, 0].reshape(T_IN, NG)
    fused = xg.T + ALPHA * dcol
    Xr = X.reshape(NG, 10, T_IN)
    Xf = jnp.concatenate([fused.T[:, None, :], Xr[:, 1:, :]], axis=1).reshape(N, T_IN)

    linw = p['lin_W'][:, 0][None, :]
    linb = p['lin_b'][None, :]
    w0t = p['conv0_W'].T
    w1t = p['conv1_W'].T
    U0_all = _embed_call(Xf.T.reshape(T_IN * N, 1), dinv2, linw, linb, w0t)

    wiht = p['gru_Wih'].T
    whht = p['gru_Whh'].T
    bih = p['gru_bih'][None, :]
    bhh = p['gru_bhh'][None, :]
    b0 = p['conv0_b'][None, :]
    b1 = p['conv1_b'][None, :]

    def step(u0, h):
        s0 = _agg(u0, src_r, dst_r)
        u1 = _mid_call(s0, dinv2, b0, w1t)
        s1 = _agg(u1, src_r, dst_r)
        return _gru_call(s1, dinv2, b1, wiht, whht, bih, bhh, h)

    h = jnp.zeros((N, HID), jnp.float32)
    for t in range(T_IN):
        h = step(U0_all[t], h)

    pred_args = (p['pred_W1'].T, p['pred_b1'][None, :], p['pred_ln1_g'][None, :],
                 p['pred_ln1_b'][None, :], p['pred_W2'].T, p['pred_b2'][None, :],
                 p['pred_ln2_g'][None, :], p['pred_ln2_b'][None, :],
                 p['pred_W3'], p['pred_b3'][None, :], linw, linb, w0t)

    outs = []
    u0 = U0_all[T_IN - 1]
    for k in range(T_OUT):
        h = step(u0, h)
        y, u0 = _pred_call(h, *pred_args, dinv2)
        outs.append(y.reshape(NG, 10)[:, 1:].reshape(-1, 1))
    return jnp.concatenate(outs, axis=1)
